# baseline (speedup 1.0000x reference)
import jax
import jax.numpy as jnp
from jax.experimental import pallas as pl
from jax.experimental.pallas import tpu as pltpu

_B, _N, _C = 8, 4096, 128
_NG = 1024
_NA = _N - _NG
_K = 1024
_HS = 64
_WS = 64
_HW = _HS * _WS
_CH = 512


def _select_body(srow_ref, scol_ref, loct_ref, out_ref):
    rank_rows = []
    for it in range(0, _NA, _CH):
        acc = jnp.zeros((1, _CH), jnp.float32)
        srow = srow_ref[0, 0:1, it:it + _CH]
        iio = jax.lax.broadcasted_iota(jnp.int32, (1, _CH), 1) + it
        for jt in range(0, _NA, _CH):
            scol = scol_ref[0, jt:jt + _CH, :]
            jio = jax.lax.broadcasted_iota(jnp.int32, (_CH, 1), 0) + jt
            gt = (scol > srow) | ((scol == srow) & (jio < iio))
            acc = acc + jnp.sum(gt.astype(jnp.float32), axis=0, keepdims=True)
        rank_rows.append(acc)
    rcol = jax.lax.broadcasted_iota(jnp.int32, (_K, 1), 0).astype(jnp.float32)
    accx = jnp.zeros((_K, 1), jnp.float32)
    accy = jnp.zeros((_K, 1), jnp.float32)
    for t, it in enumerate(range(0, _NA, _CH)):
        oh = (rank_rows[t] == rcol).astype(jnp.float32)
        lx = loct_ref[0, 0:1, it:it + _CH]
        ly = loct_ref[0, 1:2, it:it + _CH]
        accx = accx + jnp.sum(oh * lx, axis=1, keepdims=True)
        accy = accy + jnp.sum(oh * ly, axis=1, keepdims=True)
    out_ref[0, :, 0:1] = accx
    out_ref[0, :, 1:2] = accy


def _scatter_body(x_ref, loct_ref, feat_ref, cnt_ref):
    ct = pl.program_id(1)
    lx = jnp.clip(loct_ref[0, 0:1, :], -1.0, 1.0)
    ly = jnp.clip(loct_ref[0, 1:2, :], -1.0, 1.0)
    pxf = 0.5 * (lx + 1.0) * 64.0 - 0.5
    pyf = 0.5 * (ly + 1.0) * 64.0 - 0.5
    xi = jnp.clip(jnp.round(pxf).astype(jnp.int32), 0, _WS - 1)
    yi = jnp.clip(jnp.round(pyf).astype(jnp.int32), 0, _HS - 1)
    cell = xi + yi * _WS
    cell_col = jax.lax.broadcasted_iota(jnp.int32, (_CH, 1), 0) + ct * _CH
    acc = jnp.zeros((_CH, _C), jnp.float32)
    cnt = jnp.zeros((_CH, 1), jnp.float32)
    for nt in range(0, _N, _CH):
        oh = (cell[:, nt:nt + _CH] == cell_col).astype(jnp.bfloat16)
        acc = acc + jnp.dot(oh, x_ref[0, nt:nt + _CH, :].astype(jnp.bfloat16),
                            preferred_element_type=jnp.float32)
        cnt = cnt + jnp.sum(oh, axis=1, keepdims=True, dtype=jnp.float32)
    feat_ref[0] = acc
    cnt_ref[0] = cnt


_GK = None


def _gauss_weights():
    import math as _math
    import numpy as _np
    coords = _np.arange(3, dtype=_np.float32)
    x_grid = _np.tile(coords, 3).reshape(3, 3)
    y_grid = x_grid.T
    mean, variance = 1.0, 4.0
    gk = (1.0 / (2.0 * _math.pi * variance)
          * _np.exp(-((x_grid - mean) ** 2 + (y_grid - mean) ** 2)
                    / (2.0 * variance)))
    gk = gk / gk.sum()
    return gk.astype(_np.float32)


def _blur_body(feat_ref, cnt_ref, out_ref):
    gk = _gauss_weights()
    cnt = cnt_ref[0]
    mask = (cnt > 0).astype(jnp.float32)
    feature = feat_ref[0] / (cnt + 1e-6) * mask
    zf = jnp.zeros((65, _C), jnp.float32)
    zm = jnp.zeros((65, 1), jnp.float32)
    fp = jnp.concatenate([zf, feature, zf], axis=0)
    mp = jnp.concatenate([zm, mask, zm], axis=0)
    xpos = jax.lax.broadcasted_iota(jnp.int32, (_HW, 1), 0) & (_WS - 1)
    accf = jnp.zeros((_HW, _C), jnp.float32)
    accm = jnp.zeros((_HW, 1), jnp.float32)
    for dy in (-1, 0, 1):
        for dx in (-1, 0, 1):
            w = float(gk[dy + 1, dx + 1])
            o = 65 + dy * _WS + dx
            if dx == -1:
                xm = (xpos >= 1).astype(jnp.float32)
            elif dx == 1:
                xm = (xpos <= _WS - 2).astype(jnp.float32)
            else:
                xm = None
            fs = fp[o:o + _HW, :]
            ms = mp[o:o + _HW, :]
            if xm is not None:
                fs = fs * xm
                ms = ms * xm
            accf = accf + w * fs
            accm = accm + w * ms
    fi = accf / (accm + 1e-6)
    mi = (accm > 0).astype(jnp.float32)
    fi = fi * mi
    out_ref[0] = feature + (1.0 - mask) * fi


def _gather_body(xmap_ref, ld_ref, xg_ref, lg_ref, pw_ref, pb_ref, out_ref):
    pw0 = pw_ref[0:1, :]
    pw1 = pw_ref[1:2, :]
    pb = pb_ref[...]
    lgx = lg_ref[0, :, 0:1]
    lgy = lg_ref[0, :, 1:2]
    out_ref[0, 0:_NG, :] = xg_ref[0] + (lgx * pw0 + lgy * pw1 + pb)
    lx = ld_ref[0, :, 0:1]
    ly = ld_ref[0, :, 1:2]
    px = (lx + 1.0) * 0.5 * 64.0 - 0.5
    py = (ly + 1.0) * 0.5 * 64.0 - 0.5
    x0 = jnp.floor(px)
    y0 = jnp.floor(py)
    x1 = x0 + 1.0
    y1 = y0 + 1.0
    wx1 = px - x0
    wx0 = 1.0 - wx1
    wy1 = py - y0
    wy0 = 1.0 - wy1
    corners = ((x0, y0, wx0 * wy0), (x1, y0, wx1 * wy0),
               (x0, y1, wx0 * wy1), (x1, y1, wx1 * wy1))
    cws = []
    for xf, yf, w in corners:
        valid = ((xf >= 0) & (xf < _WS) & (yf >= 0) & (yf < _HS))
        xc = jnp.clip(xf, 0, _WS - 1).astype(jnp.int32)
        yc = jnp.clip(yf, 0, _HS - 1).astype(jnp.int32)
        cellc = yc * _WS + xc
        cws.append((cellc, w * valid.astype(jnp.float32)))
    acc = jnp.zeros((_K, _C), jnp.float32)
    for ct in range(0, _HW, _CH):
        ci = jax.lax.broadcasted_iota(jnp.int32, (1, _CH), 1) + ct
        oh = jnp.zeros((_K, _CH), jnp.float32)
        for cellc, w in cws:
            oh = oh + w * (cellc == ci).astype(jnp.float32)
        acc = acc + jnp.dot(oh.astype(jnp.bfloat16),
                            xmap_ref[0, ct:ct + _CH, :].astype(jnp.bfloat16),
                            preferred_element_type=jnp.float32)
    out_ref[0, _NG:, :] = acc + (lx * pw0 + ly * pw1 + pb)


def kernel(x, loc, ln_w, ln_b, W_conf, b_conf, W_pos, b_pos, H, W, N_grid):
    del H, W, N_grid
    f32 = jnp.float32
    x = x.astype(f32)
    loc = loc.astype(f32)

    u = jax.random.uniform(jax.random.key(42), (_B, _NA), dtype=f32)
    nz = -1.0 * jnp.log(u + 1e-6)
    nz = -1.0 * jnp.log(nz + 1e-6)

    loct = jnp.transpose(loc, (0, 2, 1))
    loct_ada = loct[:, :, _NG:]

    mu = jnp.mean(x, axis=-1, keepdims=True)
    var = jnp.mean((x - mu) ** 2, axis=-1, keepdims=True)
    y = (x - mu) / jnp.sqrt(var + 1e-5) * ln_w + ln_b
    conf = y @ W_conf.T + b_conf
    scores = conf[:, _NG:, 0] + nz

    loc_down = pl.pallas_call(
        _select_body,
        grid=(_B,),
        in_specs=[
            pl.BlockSpec((1, 1, _NA), lambda b: (b, 0, 0)),
            pl.BlockSpec((1, _NA, 1), lambda b: (b, 0, 0)),
            pl.BlockSpec((1, 2, _NA), lambda b: (b, 0, 0)),
        ],
        out_specs=pl.BlockSpec((1, _K, 2), lambda b: (b, 0, 0)),
        out_shape=jax.ShapeDtypeStruct((_B, _K, 2), f32),
    )(scores.reshape(_B, 1, _NA), scores.reshape(_B, _NA, 1), loct_ada)

    feat, cnt = pl.pallas_call(
        _scatter_body,
        grid=(_B, _HW // _CH),
        in_specs=[
            pl.BlockSpec((1, _N, _C), lambda b, ct: (b, 0, 0)),
            pl.BlockSpec((1, 2, _N), lambda b, ct: (b, 0, 0)),
        ],
        out_specs=[
            pl.BlockSpec((1, _CH, _C), lambda b, ct: (b, ct, 0)),
            pl.BlockSpec((1, _CH, 1), lambda b, ct: (b, ct, 0)),
        ],
        out_shape=[
            jax.ShapeDtypeStruct((_B, _HW, _C), f32),
            jax.ShapeDtypeStruct((_B, _HW, 1), f32),
        ],
    )(x, loct)

    xmap = pl.pallas_call(
        _blur_body,
        grid=(_B,),
        in_specs=[
            pl.BlockSpec((1, _HW, _C), lambda b: (b, 0, 0)),
            pl.BlockSpec((1, _HW, 1), lambda b: (b, 0, 0)),
        ],
        out_specs=pl.BlockSpec((1, _HW, _C), lambda b: (b, 0, 0)),
        out_shape=jax.ShapeDtypeStruct((_B, _HW, _C), f32),
    )(feat, cnt)

    out = pl.pallas_call(
        _gather_body,
        grid=(_B,),
        in_specs=[
            pl.BlockSpec((1, _HW, _C), lambda b: (b, 0, 0)),
            pl.BlockSpec((1, _K, 2), lambda b: (b, 0, 0)),
            pl.BlockSpec((1, _NG, _C), lambda b: (b, 0, 0)),
            pl.BlockSpec((1, _NG, 2), lambda b: (b, 0, 0)),
            pl.BlockSpec((2, _C), lambda b: (0, 0)),
            pl.BlockSpec((1, _C), lambda b: (0, 0)),
        ],
        out_specs=pl.BlockSpec((1, 2 * _K, _C), lambda b: (b, 0, 0)),
        out_shape=jax.ShapeDtypeStruct((_B, 2 * _K, _C), f32),
    )(xmap, loc_down, x[:, :_NG], loc[:, :_NG],
      jnp.transpose(W_pos.astype(f32)), b_pos.reshape(1, _C).astype(f32))

    return out


# speedup vs baseline: 1.0369x; 1.0369x over previous
import functools

import jax
import jax.numpy as jnp
from jax import lax
from jax.experimental import pallas as pl
from jax.experimental.pallas import tpu as pltpu
from jax.experimental.pallas import tpu_sc as plsc

_B, _N, _C = 8, 4096, 128
_NG = 1024
_NA = _N - _NG
_K = 1024
_HS = 64
_WS = 64
_HW = _HS * _WS
_CH = 512


def _select_body(srow_ref, scol_ref, loct_ref, out_ref):
    rank_rows = []
    for it in range(0, _NA, _CH):
        acc = jnp.zeros((1, _CH), jnp.float32)
        srow = srow_ref[0, 0:1, it:it + _CH]
        iio = jax.lax.broadcasted_iota(jnp.int32, (1, _CH), 1) + it
        for jt in range(0, _NA, _CH):
            scol = scol_ref[0, jt:jt + _CH, :]
            jio = jax.lax.broadcasted_iota(jnp.int32, (_CH, 1), 0) + jt
            gt = (scol > srow) | ((scol == srow) & (jio < iio))
            acc = acc + jnp.sum(gt.astype(jnp.float32), axis=0, keepdims=True)
        rank_rows.append(acc)
    rcol = jax.lax.broadcasted_iota(jnp.int32, (_K, 1), 0).astype(jnp.float32)
    accx = jnp.zeros((_K, 1), jnp.float32)
    accy = jnp.zeros((_K, 1), jnp.float32)
    for t, it in enumerate(range(0, _NA, _CH)):
        oh = (rank_rows[t] == rcol).astype(jnp.float32)
        lx = loct_ref[0, 0:1, it:it + _CH]
        ly = loct_ref[0, 1:2, it:it + _CH]
        accx = accx + jnp.sum(oh * lx, axis=1, keepdims=True)
        accy = accy + jnp.sum(oh * ly, axis=1, keepdims=True)
    out_ref[0, :, 0:1] = accx
    out_ref[0, :, 1:2] = accy


_TPB = _N // 16
_CPB = _HW // 16


def _cells_body(loct_ref, cells_ref, cnt_ref):
    ct = pl.program_id(1)
    lx = jnp.clip(loct_ref[0, 0:1, :], -1.0, 1.0)
    ly = jnp.clip(loct_ref[0, 1:2, :], -1.0, 1.0)
    pxf = 0.5 * (lx + 1.0) * 64.0 - 0.5
    pyf = 0.5 * (ly + 1.0) * 64.0 - 0.5
    xi = jnp.clip(jnp.round(pxf).astype(jnp.int32), 0, _WS - 1)
    yi = jnp.clip(jnp.round(pyf).astype(jnp.int32), 0, _HS - 1)
    cell = xi + yi * _WS
    cells_ref[0] = cell
    cell_col = jax.lax.broadcasted_iota(jnp.int32, (_CH, 1), 0) + ct * _CH
    cnt = jnp.zeros((_CH, 1), jnp.float32)
    for nt in range(0, _N, _CH):
        oh = (cell[:, nt:nt + _CH] == cell_col)
        cnt = cnt + jnp.sum(oh.astype(jnp.float32), axis=1, keepdims=True)
    cnt_ref[0] = cnt


def _sc_scatter_body(src_hbm, cells_hbm, zeros_hbm, out_hbm,
                     idx_v, rows_v, acc_sh):
    c = lax.axis_index("c")
    s = lax.axis_index("s")
    base = s * _TPB
    for bb in range(_B // 2):
        b = c * (_B // 2) + bb
        pltpu.sync_copy(zeros_hbm.at[pl.ds(base, _CPB)],
                        acc_sh.at[pl.ds(base, _CPB)])
        pltpu.sync_copy(src_hbm.at[b, pl.ds(base, _TPB)], rows_v)
        pltpu.sync_copy(cells_hbm.at[b, s], idx_v)
        plsc.subcore_barrier()
        for j in range(_TPB // 128):
            pltpu.sync_copy(rows_v.at[pl.ds(j * 128, 128)],
                            acc_sh.at[idx_v.at[j]], add=True)
        plsc.subcore_barrier()
        pltpu.sync_copy(acc_sh.at[pl.ds(base, _CPB)], rows_v)
        pltpu.sync_copy(rows_v, out_hbm.at[b, pl.ds(base, _CPB)])
        plsc.subcore_barrier()


_GK = None


def _gauss_weights():
    import math as _math
    import numpy as _np
    coords = _np.arange(3, dtype=_np.float32)
    x_grid = _np.tile(coords, 3).reshape(3, 3)
    y_grid = x_grid.T
    mean, variance = 1.0, 4.0
    gk = (1.0 / (2.0 * _math.pi * variance)
          * _np.exp(-((x_grid - mean) ** 2 + (y_grid - mean) ** 2)
                    / (2.0 * variance)))
    gk = gk / gk.sum()
    return gk.astype(_np.float32)


def _blur_body(feat_ref, cnt_ref, out_ref):
    gk = _gauss_weights()
    cnt = cnt_ref[0]
    mask = (cnt > 0).astype(jnp.float32)
    feature = feat_ref[0] / (cnt + 1e-6) * mask
    zf = jnp.zeros((65, _C), jnp.float32)
    zm = jnp.zeros((65, 1), jnp.float32)
    fp = jnp.concatenate([zf, feature, zf], axis=0)
    mp = jnp.concatenate([zm, mask, zm], axis=0)
    xpos = jax.lax.broadcasted_iota(jnp.int32, (_HW, 1), 0) & (_WS - 1)
    accf = jnp.zeros((_HW, _C), jnp.float32)
    accm = jnp.zeros((_HW, 1), jnp.float32)
    for dy in (-1, 0, 1):
        for dx in (-1, 0, 1):
            w = float(gk[dy + 1, dx + 1])
            o = 65 + dy * _WS + dx
            if dx == -1:
                xm = (xpos >= 1).astype(jnp.float32)
            elif dx == 1:
                xm = (xpos <= _WS - 2).astype(jnp.float32)
            else:
                xm = None
            fs = fp[o:o + _HW, :]
            ms = mp[o:o + _HW, :]
            if xm is not None:
                fs = fs * xm
                ms = ms * xm
            accf = accf + w * fs
            accm = accm + w * ms
    fi = accf / (accm + 1e-6)
    mi = (accm > 0).astype(jnp.float32)
    fi = fi * mi
    out_ref[0] = feature + (1.0 - mask) * fi


def _gather_body(xmap_ref, ld_ref, xg_ref, lg_ref, pw_ref, pb_ref, out_ref):
    pw0 = pw_ref[0:1, :]
    pw1 = pw_ref[1:2, :]
    pb = pb_ref[...]
    lgx = lg_ref[0, :, 0:1]
    lgy = lg_ref[0, :, 1:2]
    out_ref[0, 0:_NG, :] = xg_ref[0] + (lgx * pw0 + lgy * pw1 + pb)
    lx = ld_ref[0, :, 0:1]
    ly = ld_ref[0, :, 1:2]
    px = (lx + 1.0) * 0.5 * 64.0 - 0.5
    py = (ly + 1.0) * 0.5 * 64.0 - 0.5
    x0 = jnp.floor(px)
    y0 = jnp.floor(py)
    x1 = x0 + 1.0
    y1 = y0 + 1.0
    wx1 = px - x0
    wx0 = 1.0 - wx1
    wy1 = py - y0
    wy0 = 1.0 - wy1
    corners = ((x0, y0, wx0 * wy0), (x1, y0, wx1 * wy0),
               (x0, y1, wx0 * wy1), (x1, y1, wx1 * wy1))
    cws = []
    for xf, yf, w in corners:
        valid = ((xf >= 0) & (xf < _WS) & (yf >= 0) & (yf < _HS))
        xc = jnp.clip(xf, 0, _WS - 1).astype(jnp.int32)
        yc = jnp.clip(yf, 0, _HS - 1).astype(jnp.int32)
        cellc = yc * _WS + xc
        cws.append((cellc, w * valid.astype(jnp.float32)))
    acc = jnp.zeros((_K, _C), jnp.float32)
    for ct in range(0, _HW, _CH):
        ci = jax.lax.broadcasted_iota(jnp.int32, (1, _CH), 1) + ct
        oh = jnp.zeros((_K, _CH), jnp.float32)
        for cellc, w in cws:
            oh = oh + w * (cellc == ci).astype(jnp.float32)
        acc = acc + jnp.dot(oh.astype(jnp.bfloat16),
                            xmap_ref[0, ct:ct + _CH, :].astype(jnp.bfloat16),
                            preferred_element_type=jnp.float32)
    out_ref[0, _NG:, :] = acc + (lx * pw0 + ly * pw1 + pb)


def kernel(x, loc, ln_w, ln_b, W_conf, b_conf, W_pos, b_pos, H, W, N_grid):
    del H, W, N_grid
    f32 = jnp.float32
    x = x.astype(f32)
    loc = loc.astype(f32)

    u = jax.random.uniform(jax.random.key(42), (_B, _NA), dtype=f32)
    nz = -1.0 * jnp.log(u + 1e-6)
    nz = -1.0 * jnp.log(nz + 1e-6)

    loct = jnp.transpose(loc, (0, 2, 1))
    loct_ada = loct[:, :, _NG:]

    mu = jnp.mean(x, axis=-1, keepdims=True)
    var = jnp.mean((x - mu) ** 2, axis=-1, keepdims=True)
    y = (x - mu) / jnp.sqrt(var + 1e-5) * ln_w + ln_b
    conf = y @ W_conf.T + b_conf
    scores = conf[:, _NG:, 0] + nz

    loc_down = pl.pallas_call(
        _select_body,
        grid=(_B,),
        in_specs=[
            pl.BlockSpec((1, 1, _NA), lambda b: (b, 0, 0)),
            pl.BlockSpec((1, _NA, 1), lambda b: (b, 0, 0)),
            pl.BlockSpec((1, 2, _NA), lambda b: (b, 0, 0)),
        ],
        out_specs=pl.BlockSpec((1, _K, 2), lambda b: (b, 0, 0)),
        out_shape=jax.ShapeDtypeStruct((_B, _K, 2), f32),
    )(scores.reshape(_B, 1, _NA), scores.reshape(_B, _NA, 1), loct_ada)

    cells, cnt = pl.pallas_call(
        _cells_body,
        grid=(_B, _HW // _CH),
        in_specs=[pl.BlockSpec((1, 2, _N), lambda b, ct: (b, 0, 0))],
        out_specs=[
            pl.BlockSpec((1, 1, _N), lambda b, ct: (b, 0, 0)),
            pl.BlockSpec((1, _CH, 1), lambda b, ct: (b, ct, 0)),
        ],
        out_shape=[
            jax.ShapeDtypeStruct((_B, 1, _N), jnp.int32),
            jax.ShapeDtypeStruct((_B, _HW, 1), f32),
        ],
    )(loct)

    sc_scatter = functools.partial(
        pl.kernel,
        out_type=jax.ShapeDtypeStruct((_B, _HW, _C), f32),
        mesh=plsc.VectorSubcoreMesh(core_axis_name="c", subcore_axis_name="s"),
        scratch_types=[
            pltpu.VMEM((_TPB // 128, 128), jnp.int32),
            pltpu.VMEM((_TPB, _C), f32),
            pltpu.VMEM_SHARED((_HW, _C), f32),
        ],
    )(_sc_scatter_body)
    feat = sc_scatter(x, cells.reshape(_B, 16, _TPB // 128, 128),
                      jnp.zeros((_HW, _C), f32))

    xmap = pl.pallas_call(
        _blur_body,
        grid=(_B,),
        in_specs=[
            pl.BlockSpec((1, _HW, _C), lambda b: (b, 0, 0)),
            pl.BlockSpec((1, _HW, 1), lambda b: (b, 0, 0)),
        ],
        out_specs=pl.BlockSpec((1, _HW, _C), lambda b: (b, 0, 0)),
        out_shape=jax.ShapeDtypeStruct((_B, _HW, _C), f32),
    )(feat, cnt)

    out = pl.pallas_call(
        _gather_body,
        grid=(_B,),
        in_specs=[
            pl.BlockSpec((1, _HW, _C), lambda b: (b, 0, 0)),
            pl.BlockSpec((1, _K, 2), lambda b: (b, 0, 0)),
            pl.BlockSpec((1, _NG, _C), lambda b: (b, 0, 0)),
            pl.BlockSpec((1, _NG, 2), lambda b: (b, 0, 0)),
            pl.BlockSpec((2, _C), lambda b: (0, 0)),
            pl.BlockSpec((1, _C), lambda b: (0, 0)),
        ],
        out_specs=pl.BlockSpec((1, 2 * _K, _C), lambda b: (b, 0, 0)),
        out_shape=jax.ShapeDtypeStruct((_B, 2 * _K, _C), f32),
    )(xmap, loc_down, x[:, :_NG], loc[:, :_NG],
      jnp.transpose(W_pos.astype(f32)), b_pos.reshape(1, _C).astype(f32))

    return out


# speedup vs baseline: 1.1621x; 1.1207x over previous
import functools

import jax
import jax.numpy as jnp
from jax import lax
from jax.experimental import pallas as pl
from jax.experimental.pallas import tpu as pltpu
from jax.experimental.pallas import tpu_sc as plsc

_B, _N, _C = 8, 4096, 128
_NG = 1024
_NA = _N - _NG
_K = 1024
_HS = 64
_WS = 64
_HW = _HS * _WS
_CH = 512


def _select_body(srow_ref, scol_ref, loct_ref, out_ref, cidx_ref, w4_ref):
    rank_rows = []
    for it in range(0, _NA, _CH):
        acc = jnp.zeros((1, _CH), jnp.float32)
        srow = srow_ref[0, 0:1, it:it + _CH]
        iio = jax.lax.broadcasted_iota(jnp.int32, (1, _CH), 1) + it
        for jt in range(0, _NA, _CH):
            scol = scol_ref[0, jt:jt + _CH, :]
            jio = jax.lax.broadcasted_iota(jnp.int32, (_CH, 1), 0) + jt
            gt = (scol > srow) | ((scol == srow) & (jio < iio))
            acc = acc + jnp.sum(gt.astype(jnp.float32), axis=0, keepdims=True)
        rank_rows.append(acc)
    rcol = jax.lax.broadcasted_iota(jnp.int32, (_K, 1), 0).astype(jnp.float32)
    accx = jnp.zeros((_K, 1), jnp.float32)
    accy = jnp.zeros((_K, 1), jnp.float32)
    for t, it in enumerate(range(0, _NA, _CH)):
        oh = (rank_rows[t] == rcol).astype(jnp.float32)
        lx = loct_ref[0, 0:1, it:it + _CH]
        ly = loct_ref[0, 1:2, it:it + _CH]
        accx = accx + jnp.sum(oh * lx, axis=1, keepdims=True)
        accy = accy + jnp.sum(oh * ly, axis=1, keepdims=True)
    out_ref[0, :, 0:1] = accx
    out_ref[0, :, 1:2] = accy
    px = (accx + 1.0) * 0.5 * 64.0 - 0.5
    py = (accy + 1.0) * 0.5 * 64.0 - 0.5
    x0 = jnp.floor(px)
    y0 = jnp.floor(py)
    wx1 = px - x0
    wx0 = 1.0 - wx1
    wy1 = py - y0
    wy0 = 1.0 - wy1
    corners = ((x0, y0, wx0 * wy0), (x0 + 1.0, y0, wx1 * wy0),
               (x0, y0 + 1.0, wx0 * wy1), (x0 + 1.0, y0 + 1.0, wx1 * wy1))
    for ci, (xf, yf, w) in enumerate(corners):
        valid = ((xf >= 0) & (xf < _WS) & (yf >= 0) & (yf < _HS))
        w4_ref[0, :, ci:ci + 1] = w * valid.astype(jnp.float32)
        xc = jnp.clip(xf, 0, _WS - 1).astype(jnp.int32)
        yc = jnp.clip(yf, 0, _HS - 1).astype(jnp.int32)
        cidx_ref[0, ci, :] = jnp.reshape(yc * _WS + xc, (_K,))


_TPB = _N // 16
_CPB = _HW // 16


def _cells_body(loct_ref, cells_ref, cnt_ref):
    ct = pl.program_id(1)
    lx = jnp.clip(loct_ref[0, 0:1, :], -1.0, 1.0)
    ly = jnp.clip(loct_ref[0, 1:2, :], -1.0, 1.0)
    pxf = 0.5 * (lx + 1.0) * 64.0 - 0.5
    pyf = 0.5 * (ly + 1.0) * 64.0 - 0.5
    xi = jnp.clip(jnp.round(pxf).astype(jnp.int32), 0, _WS - 1)
    yi = jnp.clip(jnp.round(pyf).astype(jnp.int32), 0, _HS - 1)
    cell = xi + yi * _WS
    cells_ref[0] = cell
    cell_col = jax.lax.broadcasted_iota(jnp.int32, (_CH, 1), 0) + ct * _CH
    cnt = jnp.zeros((_CH, 1), jnp.float32)
    for nt in range(0, _N, _CH):
        oh = (cell[:, nt:nt + _CH] == cell_col)
        cnt = cnt + jnp.sum(oh.astype(jnp.float32), axis=1, keepdims=True)
    cnt_ref[0] = cnt


def _sc_scatter_body(src_hbm, cells_hbm, zeros_hbm, out_hbm,
                     idx_v, rows_v, acc_sh):
    c = lax.axis_index("c")
    s = lax.axis_index("s")
    base = s * _TPB
    for bb in range(_B // 2):
        b = c * (_B // 2) + bb
        pltpu.sync_copy(zeros_hbm.at[pl.ds(base, _CPB)],
                        acc_sh.at[pl.ds(base, _CPB)])
        pltpu.sync_copy(src_hbm.at[b, pl.ds(base, _TPB)], rows_v)
        pltpu.sync_copy(cells_hbm.at[b, s], idx_v)
        plsc.subcore_barrier()
        for j in range(_TPB // 128):
            pltpu.sync_copy(rows_v.at[pl.ds(j * 128, 128)],
                            acc_sh.at[idx_v.at[j]], add=True)
        plsc.subcore_barrier()
        pltpu.sync_copy(acc_sh.at[pl.ds(base, _CPB)], rows_v)
        pltpu.sync_copy(rows_v, out_hbm.at[b, pl.ds(base, _CPB)])
        plsc.subcore_barrier()


_GK = None


def _gauss_weights():
    import math as _math
    import numpy as _np
    coords = _np.arange(3, dtype=_np.float32)
    x_grid = _np.tile(coords, 3).reshape(3, 3)
    y_grid = x_grid.T
    mean, variance = 1.0, 4.0
    gk = (1.0 / (2.0 * _math.pi * variance)
          * _np.exp(-((x_grid - mean) ** 2 + (y_grid - mean) ** 2)
                    / (2.0 * variance)))
    gk = gk / gk.sum()
    return gk.astype(_np.float32)


def _blur_body(feat_ref, cnt_ref, out_ref):
    gk = _gauss_weights()
    cnt = cnt_ref[0]
    mask = (cnt > 0).astype(jnp.float32)
    feature = feat_ref[0] / (cnt + 1e-6) * mask
    zf = jnp.zeros((65, _C), jnp.float32)
    zm = jnp.zeros((65, 1), jnp.float32)
    fp = jnp.concatenate([zf, feature, zf], axis=0)
    mp = jnp.concatenate([zm, mask, zm], axis=0)
    xpos = jax.lax.broadcasted_iota(jnp.int32, (_HW, 1), 0) & (_WS - 1)
    accf = jnp.zeros((_HW, _C), jnp.float32)
    accm = jnp.zeros((_HW, 1), jnp.float32)
    for dy in (-1, 0, 1):
        for dx in (-1, 0, 1):
            w = float(gk[dy + 1, dx + 1])
            o = 65 + dy * _WS + dx
            if dx == -1:
                xm = (xpos >= 1).astype(jnp.float32)
            elif dx == 1:
                xm = (xpos <= _WS - 2).astype(jnp.float32)
            else:
                xm = None
            fs = fp[o:o + _HW, :]
            ms = mp[o:o + _HW, :]
            if xm is not None:
                fs = fs * xm
                ms = ms * xm
            accf = accf + w * fs
            accm = accm + w * ms
    fi = accf / (accm + 1e-6)
    mi = (accm > 0).astype(jnp.float32)
    fi = fi * mi
    out_ref[0] = feature + (1.0 - mask) * fi


def _sc_gather_body(xmap_hbm, cidx_hbm, out_hbm, idx_v, rows_v, sem):
    c = lax.axis_index("c")
    s = lax.axis_index("s")
    base = s * _TPB
    for bb in range(_B // 2):
        b = c * (_B // 2) + bb
        pltpu.sync_copy(cidx_hbm.at[b, s], idx_v)
        for j in range(_TPB // 128):
            pltpu.async_copy(xmap_hbm.at[b].at[idx_v.at[j]],
                             rows_v.at[pl.ds(j * 128, 128)], sem).wait()
        pltpu.sync_copy(rows_v, out_hbm.at[b, pl.ds(base, _TPB)])


def _combine_body(rows_ref, w4_ref, ld_ref, xg_ref, lg_ref, pw_ref, pb_ref,
                  out_ref):
    pw0 = pw_ref[0:1, :]
    pw1 = pw_ref[1:2, :]
    pb = pb_ref[...]
    lgx = lg_ref[0, :, 0:1]
    lgy = lg_ref[0, :, 1:2]
    out_ref[0, 0:_NG, :] = xg_ref[0] + (lgx * pw0 + lgy * pw1 + pb)
    acc = jnp.zeros((_K, _C), jnp.float32)
    for ci in range(4):
        acc = acc + w4_ref[0, :, ci:ci + 1] * rows_ref[0, ci]
    lx = ld_ref[0, :, 0:1]
    ly = ld_ref[0, :, 1:2]
    out_ref[0, _NG:, :] = acc + (lx * pw0 + ly * pw1 + pb)


def kernel(x, loc, ln_w, ln_b, W_conf, b_conf, W_pos, b_pos, H, W, N_grid):
    del H, W, N_grid
    f32 = jnp.float32
    x = x.astype(f32)
    loc = loc.astype(f32)

    u = jax.random.uniform(jax.random.key(42), (_B, _NA), dtype=f32)
    nz = -1.0 * jnp.log(u + 1e-6)
    nz = -1.0 * jnp.log(nz + 1e-6)

    loct = jnp.transpose(loc, (0, 2, 1))
    loct_ada = loct[:, :, _NG:]

    mu = jnp.mean(x, axis=-1, keepdims=True)
    var = jnp.mean((x - mu) ** 2, axis=-1, keepdims=True)
    y = (x - mu) / jnp.sqrt(var + 1e-5) * ln_w + ln_b
    conf = y @ W_conf.T + b_conf
    scores = conf[:, _NG:, 0] + nz

    loc_down, cidx, w4 = pl.pallas_call(
        _select_body,
        grid=(_B,),
        in_specs=[
            pl.BlockSpec((1, 1, _NA), lambda b: (b, 0, 0)),
            pl.BlockSpec((1, _NA, 1), lambda b: (b, 0, 0)),
            pl.BlockSpec((1, 2, _NA), lambda b: (b, 0, 0)),
        ],
        out_specs=[
            pl.BlockSpec((1, _K, 2), lambda b: (b, 0, 0)),
            pl.BlockSpec((1, 4, _K), lambda b: (b, 0, 0)),
            pl.BlockSpec((1, _K, 4), lambda b: (b, 0, 0)),
        ],
        out_shape=[
            jax.ShapeDtypeStruct((_B, _K, 2), f32),
            jax.ShapeDtypeStruct((_B, 4, _K), jnp.int32),
            jax.ShapeDtypeStruct((_B, _K, 4), f32),
        ],
    )(scores.reshape(_B, 1, _NA), scores.reshape(_B, _NA, 1), loct_ada)

    cells, cnt = pl.pallas_call(
        _cells_body,
        grid=(_B, _HW // _CH),
        in_specs=[pl.BlockSpec((1, 2, _N), lambda b, ct: (b, 0, 0))],
        out_specs=[
            pl.BlockSpec((1, 1, _N), lambda b, ct: (b, 0, 0)),
            pl.BlockSpec((1, _CH, 1), lambda b, ct: (b, ct, 0)),
        ],
        out_shape=[
            jax.ShapeDtypeStruct((_B, 1, _N), jnp.int32),
            jax.ShapeDtypeStruct((_B, _HW, 1), f32),
        ],
    )(loct)

    sc_scatter = functools.partial(
        pl.kernel,
        out_type=jax.ShapeDtypeStruct((_B, _HW, _C), f32),
        mesh=plsc.VectorSubcoreMesh(core_axis_name="c", subcore_axis_name="s"),
        scratch_types=[
            pltpu.VMEM((_TPB // 128, 128), jnp.int32),
            pltpu.VMEM((_TPB, _C), f32),
            pltpu.VMEM_SHARED((_HW, _C), f32),
        ],
    )(_sc_scatter_body)
    feat = sc_scatter(x, cells.reshape(_B, 16, _TPB // 128, 128),
                      jnp.zeros((_HW, _C), f32))

    xmap = pl.pallas_call(
        _blur_body,
        grid=(_B,),
        in_specs=[
            pl.BlockSpec((1, _HW, _C), lambda b: (b, 0, 0)),
            pl.BlockSpec((1, _HW, 1), lambda b: (b, 0, 0)),
        ],
        out_specs=pl.BlockSpec((1, _HW, _C), lambda b: (b, 0, 0)),
        out_shape=jax.ShapeDtypeStruct((_B, _HW, _C), f32),
    )(feat, cnt)

    sc_gather = functools.partial(
        pl.kernel,
        out_type=jax.ShapeDtypeStruct((_B, 4 * _K, _C), f32),
        mesh=plsc.VectorSubcoreMesh(core_axis_name="c", subcore_axis_name="s"),
        scratch_types=[
            pltpu.VMEM((_TPB // 128, 128), jnp.int32),
            pltpu.VMEM((_TPB, _C), f32),
            pltpu.SemaphoreType.DMA,
        ],
    )(_sc_gather_body)
    rows4 = sc_gather(xmap, cidx.reshape(_B, 16, _TPB // 128, 128))

    out = pl.pallas_call(
        _combine_body,
        grid=(_B,),
        in_specs=[
            pl.BlockSpec((1, 4, _K, _C), lambda b: (b, 0, 0, 0)),
            pl.BlockSpec((1, _K, 4), lambda b: (b, 0, 0)),
            pl.BlockSpec((1, _K, 2), lambda b: (b, 0, 0)),
            pl.BlockSpec((1, _NG, _C), lambda b: (b, 0, 0)),
            pl.BlockSpec((1, _NG, 2), lambda b: (b, 0, 0)),
            pl.BlockSpec((2, _C), lambda b: (0, 0)),
            pl.BlockSpec((1, _C), lambda b: (0, 0)),
        ],
        out_specs=pl.BlockSpec((1, 2 * _K, _C), lambda b: (b, 0, 0)),
        out_shape=jax.ShapeDtypeStruct((_B, 2 * _K, _C), f32),
    )(rows4.reshape(_B, 4, _K, _C), w4, loc_down, x[:, :_NG], loc[:, :_NG],
      jnp.transpose(W_pos.astype(f32)), b_pos.reshape(1, _C).astype(f32))

    return out


# speedup vs baseline: 1.1643x; 1.0019x over previous
import functools

import jax
import jax.numpy as jnp
from jax import lax
from jax.experimental import pallas as pl
from jax.experimental.pallas import tpu as pltpu
from jax.experimental.pallas import tpu_sc as plsc

_B, _N, _C = 8, 4096, 128
_NG = 1024
_NA = _N - _NG
_K = 1024
_HS = 64
_WS = 64
_HW = _HS * _WS
_CH = 512


def _select_body(srow_ref, scol_ref, loct_ref, out_ref, cidx_ref, w4_ref):
    rank_rows = []
    for it in range(0, _NA, _CH):
        acc = jnp.zeros((1, _CH), jnp.float32)
        srow = srow_ref[0, 0:1, it:it + _CH]
        iio = jax.lax.broadcasted_iota(jnp.int32, (1, _CH), 1) + it
        for jt in range(0, _NA, _CH):
            scol = scol_ref[0, jt:jt + _CH, :]
            jio = jax.lax.broadcasted_iota(jnp.int32, (_CH, 1), 0) + jt
            gt = (scol > srow) | ((scol == srow) & (jio < iio))
            acc = acc + jnp.sum(gt.astype(jnp.float32), axis=0, keepdims=True)
        rank_rows.append(acc)
    rcol = jax.lax.broadcasted_iota(jnp.int32, (_K, 1), 0).astype(jnp.float32)
    accx = jnp.zeros((_K, 1), jnp.float32)
    accy = jnp.zeros((_K, 1), jnp.float32)
    for t, it in enumerate(range(0, _NA, _CH)):
        oh = (rank_rows[t] == rcol).astype(jnp.float32)
        lx = loct_ref[0, 0:1, it:it + _CH]
        ly = loct_ref[0, 1:2, it:it + _CH]
        accx = accx + jnp.sum(oh * lx, axis=1, keepdims=True)
        accy = accy + jnp.sum(oh * ly, axis=1, keepdims=True)
    out_ref[0, :, 0:1] = accx
    out_ref[0, :, 1:2] = accy
    px = (accx + 1.0) * 0.5 * 64.0 - 0.5
    py = (accy + 1.0) * 0.5 * 64.0 - 0.5
    x0 = jnp.floor(px)
    y0 = jnp.floor(py)
    wx1 = px - x0
    wx0 = 1.0 - wx1
    wy1 = py - y0
    wy0 = 1.0 - wy1
    corners = ((x0, y0, wx0 * wy0), (x0 + 1.0, y0, wx1 * wy0),
               (x0, y0 + 1.0, wx0 * wy1), (x0 + 1.0, y0 + 1.0, wx1 * wy1))
    for ci, (xf, yf, w) in enumerate(corners):
        valid = ((xf >= 0) & (xf < _WS) & (yf >= 0) & (yf < _HS))
        w4_ref[0, :, ci:ci + 1] = w * valid.astype(jnp.float32)
        xc = jnp.clip(xf, 0, _WS - 1).astype(jnp.int32)
        yc = jnp.clip(yf, 0, _HS - 1).astype(jnp.int32)
        cidx_ref[0, ci, :] = jnp.reshape(yc * _WS + xc, (_K,))


_TPB = _N // 16
_CPB = _HW // 16


def _cells_body(loct_ref, cells_ref):
    lx = jnp.clip(loct_ref[0, 0:1, :], -1.0, 1.0)
    ly = jnp.clip(loct_ref[0, 1:2, :], -1.0, 1.0)
    pxf = 0.5 * (lx + 1.0) * 64.0 - 0.5
    pyf = 0.5 * (ly + 1.0) * 64.0 - 0.5
    xi = jnp.clip(jnp.round(pxf).astype(jnp.int32), 0, _WS - 1)
    yi = jnp.clip(jnp.round(pyf).astype(jnp.int32), 0, _HS - 1)
    cells_ref[0] = xi + yi * _WS


def _hist_body(cells_ref, cnt_ref):
    ct = pl.program_id(1)
    cell = cells_ref[0]
    cell_col = jax.lax.broadcasted_iota(jnp.int32, (_CH, 1), 0) + ct * _CH
    cnt = jnp.zeros((_CH, 1), jnp.float32)
    for nt in range(0, _N, _CH):
        oh = (cell[:, nt:nt + _CH] == cell_col)
        cnt = cnt + jnp.sum(oh.astype(jnp.float32), axis=1, keepdims=True)
    cnt_ref[0] = cnt


def _sc_scatter_body(src_hbm, cells_hbm, zeros_hbm, out_hbm,
                     idx_v, rows_v, acc_sh):
    c = lax.axis_index("c")
    s = lax.axis_index("s")
    base = s * _TPB
    for bb in range(_B // 2):
        b = c * (_B // 2) + bb
        pltpu.sync_copy(zeros_hbm.at[pl.ds(base, _CPB)],
                        acc_sh.at[pl.ds(base, _CPB)])
        pltpu.sync_copy(src_hbm.at[b, pl.ds(base, _TPB)], rows_v)
        pltpu.sync_copy(cells_hbm.at[b, s], idx_v)
        plsc.subcore_barrier()
        for j in range(_TPB // 128):
            pltpu.sync_copy(rows_v.at[pl.ds(j * 128, 128)],
                            acc_sh.at[idx_v.at[j]], add=True)
        plsc.subcore_barrier()
        pltpu.sync_copy(acc_sh.at[pl.ds(base, _CPB)], rows_v)
        pltpu.sync_copy(rows_v, out_hbm.at[b, pl.ds(base, _CPB)])
        plsc.subcore_barrier()


_GK = None


def _gauss_weights():
    import math as _math
    import numpy as _np
    coords = _np.arange(3, dtype=_np.float32)
    x_grid = _np.tile(coords, 3).reshape(3, 3)
    y_grid = x_grid.T
    mean, variance = 1.0, 4.0
    gk = (1.0 / (2.0 * _math.pi * variance)
          * _np.exp(-((x_grid - mean) ** 2 + (y_grid - mean) ** 2)
                    / (2.0 * variance)))
    gk = gk / gk.sum()
    return gk.astype(_np.float32)


def _blur_body(feat_ref, cnt_ref, out_ref):
    gk = _gauss_weights()
    cnt = cnt_ref[0]
    mask = (cnt > 0).astype(jnp.float32)
    feature = feat_ref[0] / (cnt + 1e-6) * mask
    zf = jnp.zeros((65, _C), jnp.float32)
    zm = jnp.zeros((65, 1), jnp.float32)
    fp = jnp.concatenate([zf, feature, zf], axis=0)
    mp = jnp.concatenate([zm, mask, zm], axis=0)
    xpos = jax.lax.broadcasted_iota(jnp.int32, (_HW, 1), 0) & (_WS - 1)
    accf = jnp.zeros((_HW, _C), jnp.float32)
    accm = jnp.zeros((_HW, 1), jnp.float32)
    for dy in (-1, 0, 1):
        for dx in (-1, 0, 1):
            w = float(gk[dy + 1, dx + 1])
            o = 65 + dy * _WS + dx
            if dx == -1:
                xm = (xpos >= 1).astype(jnp.float32)
            elif dx == 1:
                xm = (xpos <= _WS - 2).astype(jnp.float32)
            else:
                xm = None
            fs = fp[o:o + _HW, :]
            ms = mp[o:o + _HW, :]
            if xm is not None:
                fs = fs * xm
                ms = ms * xm
            accf = accf + w * fs
            accm = accm + w * ms
    fi = accf / (accm + 1e-6)
    mi = (accm > 0).astype(jnp.float32)
    fi = fi * mi
    out_ref[0] = feature + (1.0 - mask) * fi


def _sc_gather_body(xmap_hbm, cidx_hbm, out_hbm, idx_v, rows_v, sem):
    c = lax.axis_index("c")
    s = lax.axis_index("s")
    base = s * _TPB
    for bb in range(_B // 2):
        b = c * (_B // 2) + bb
        pltpu.sync_copy(cidx_hbm.at[b, s], idx_v)
        for j in range(_TPB // 128):
            pltpu.async_copy(xmap_hbm.at[b].at[idx_v.at[j]],
                             rows_v.at[pl.ds(j * 128, 128)], sem).wait()
        pltpu.sync_copy(rows_v, out_hbm.at[b, pl.ds(base, _TPB)])


def _combine_body(rows_ref, w4_ref, ld_ref, xg_ref, lg_ref, pw_ref, pb_ref,
                  out_ref):
    pw0 = pw_ref[0:1, :]
    pw1 = pw_ref[1:2, :]
    pb = pb_ref[...]
    lgx = lg_ref[0, :, 0:1]
    lgy = lg_ref[0, :, 1:2]
    out_ref[0, 0:_NG, :] = xg_ref[0] + (lgx * pw0 + lgy * pw1 + pb)
    acc = jnp.zeros((_K, _C), jnp.float32)
    for ci in range(4):
        acc = acc + w4_ref[0, :, ci:ci + 1] * rows_ref[0, ci]
    lx = ld_ref[0, :, 0:1]
    ly = ld_ref[0, :, 1:2]
    out_ref[0, _NG:, :] = acc + (lx * pw0 + ly * pw1 + pb)


def kernel(x, loc, ln_w, ln_b, W_conf, b_conf, W_pos, b_pos, H, W, N_grid):
    del H, W, N_grid
    f32 = jnp.float32
    x = x.astype(f32)
    loc = loc.astype(f32)

    u = jax.random.uniform(jax.random.key(42), (_B, _NA), dtype=f32)
    nz = -1.0 * jnp.log(u + 1e-6)
    nz = -1.0 * jnp.log(nz + 1e-6)

    loct = jnp.transpose(loc, (0, 2, 1))
    loct_ada = loct[:, :, _NG:]

    mu = jnp.mean(x, axis=-1, keepdims=True)
    var = jnp.mean((x - mu) ** 2, axis=-1, keepdims=True)
    y = (x - mu) / jnp.sqrt(var + 1e-5) * ln_w + ln_b
    conf = y @ W_conf.T + b_conf
    scores = conf[:, _NG:, 0] + nz

    loc_down, cidx, w4 = pl.pallas_call(
        _select_body,
        grid=(_B,),
        in_specs=[
            pl.BlockSpec((1, 1, _NA), lambda b: (b, 0, 0)),
            pl.BlockSpec((1, _NA, 1), lambda b: (b, 0, 0)),
            pl.BlockSpec((1, 2, _NA), lambda b: (b, 0, 0)),
        ],
        out_specs=[
            pl.BlockSpec((1, _K, 2), lambda b: (b, 0, 0)),
            pl.BlockSpec((1, 4, _K), lambda b: (b, 0, 0)),
            pl.BlockSpec((1, _K, 4), lambda b: (b, 0, 0)),
        ],
        out_shape=[
            jax.ShapeDtypeStruct((_B, _K, 2), f32),
            jax.ShapeDtypeStruct((_B, 4, _K), jnp.int32),
            jax.ShapeDtypeStruct((_B, _K, 4), f32),
        ],
    )(scores.reshape(_B, 1, _NA), scores.reshape(_B, _NA, 1), loct_ada)

    cells = pl.pallas_call(
        _cells_body,
        grid=(_B,),
        in_specs=[pl.BlockSpec((1, 2, _N), lambda b: (b, 0, 0))],
        out_specs=pl.BlockSpec((1, 1, _N), lambda b: (b, 0, 0)),
        out_shape=jax.ShapeDtypeStruct((_B, 1, _N), jnp.int32),
    )(loct)

    cnt = pl.pallas_call(
        _hist_body,
        grid=(_B, _HW // _CH),
        in_specs=[pl.BlockSpec((1, 1, _N), lambda b, ct: (b, 0, 0))],
        out_specs=pl.BlockSpec((1, _CH, 1), lambda b, ct: (b, ct, 0)),
        out_shape=jax.ShapeDtypeStruct((_B, _HW, 1), f32),
    )(cells)

    sc_scatter = functools.partial(
        pl.kernel,
        out_type=jax.ShapeDtypeStruct((_B, _HW, _C), f32),
        mesh=plsc.VectorSubcoreMesh(core_axis_name="c", subcore_axis_name="s"),
        scratch_types=[
            pltpu.VMEM((_TPB // 128, 128), jnp.int32),
            pltpu.VMEM((_TPB, _C), f32),
            pltpu.VMEM_SHARED((_HW, _C), f32),
        ],
    )(_sc_scatter_body)
    feat = sc_scatter(x, cells.reshape(_B, 16, _TPB // 128, 128),
                      jnp.zeros((_HW, _C), f32))

    xmap = pl.pallas_call(
        _blur_body,
        grid=(_B,),
        in_specs=[
            pl.BlockSpec((1, _HW, _C), lambda b: (b, 0, 0)),
            pl.BlockSpec((1, _HW, 1), lambda b: (b, 0, 0)),
        ],
        out_specs=pl.BlockSpec((1, _HW, _C), lambda b: (b, 0, 0)),
        out_shape=jax.ShapeDtypeStruct((_B, _HW, _C), f32),
    )(feat, cnt)

    sc_gather = functools.partial(
        pl.kernel,
        out_type=jax.ShapeDtypeStruct((_B, 4 * _K, _C), f32),
        mesh=plsc.VectorSubcoreMesh(core_axis_name="c", subcore_axis_name="s"),
        scratch_types=[
            pltpu.VMEM((_TPB // 128, 128), jnp.int32),
            pltpu.VMEM((_TPB, _C), f32),
            pltpu.SemaphoreType.DMA,
        ],
    )(_sc_gather_body)
    rows4 = sc_gather(xmap, cidx.reshape(_B, 16, _TPB // 128, 128))

    out = pl.pallas_call(
        _combine_body,
        grid=(_B,),
        in_specs=[
            pl.BlockSpec((1, 4, _K, _C), lambda b: (b, 0, 0, 0)),
            pl.BlockSpec((1, _K, 4), lambda b: (b, 0, 0)),
            pl.BlockSpec((1, _K, 2), lambda b: (b, 0, 0)),
            pl.BlockSpec((1, _NG, _C), lambda b: (b, 0, 0)),
            pl.BlockSpec((1, _NG, 2), lambda b: (b, 0, 0)),
            pl.BlockSpec((2, _C), lambda b: (0, 0)),
            pl.BlockSpec((1, _C), lambda b: (0, 0)),
        ],
        out_specs=pl.BlockSpec((1, 2 * _K, _C), lambda b: (b, 0, 0)),
        out_shape=jax.ShapeDtypeStruct((_B, 2 * _K, _C), f32),
    )(rows4.reshape(_B, 4, _K, _C), w4, loc_down, x[:, :_NG], loc[:, :_NG],
      jnp.transpose(W_pos.astype(f32)), b_pos.reshape(1, _C).astype(f32))

    return out


# speedup vs baseline: 1.2195x; 1.0473x over previous
import functools

import jax
import jax.numpy as jnp
from jax import lax
from jax.experimental import pallas as pl
from jax.experimental.pallas import tpu as pltpu
from jax.experimental.pallas import tpu_sc as plsc

_B, _N, _C = 8, 4096, 128
_NG = 1024
_NA = _N - _NG
_K = 1024
_HS = 64
_WS = 64
_HW = _HS * _WS
_CH = 512


def _select_body(srow_ref, scol_ref, loct_ref, out_ref, cidx_ref, w4_ref):
    rank_rows = []
    for it in range(0, _NA, _CH):
        acc = jnp.zeros((1, _CH), jnp.float32)
        srow = srow_ref[0, 0:1, it:it + _CH]
        iio = jax.lax.broadcasted_iota(jnp.int32, (1, _CH), 1) + it
        for jt in range(0, _NA, _CH):
            scol = scol_ref[0, jt:jt + _CH, :]
            jio = jax.lax.broadcasted_iota(jnp.int32, (_CH, 1), 0) + jt
            gt = (scol > srow) | ((scol == srow) & (jio < iio))
            acc = acc + jnp.sum(gt.astype(jnp.float32), axis=0, keepdims=True)
        rank_rows.append(acc)
    rcol = jax.lax.broadcasted_iota(jnp.int32, (_K, 1), 0).astype(jnp.float32)
    accx = jnp.zeros((_K, 1), jnp.float32)
    accy = jnp.zeros((_K, 1), jnp.float32)
    for t, it in enumerate(range(0, _NA, _CH)):
        oh = (rank_rows[t] == rcol).astype(jnp.float32)
        lx = loct_ref[0, 0:1, it:it + _CH]
        ly = loct_ref[0, 1:2, it:it + _CH]
        accx = accx + jnp.sum(oh * lx, axis=1, keepdims=True)
        accy = accy + jnp.sum(oh * ly, axis=1, keepdims=True)
    out_ref[0, :, 0:1] = accx
    out_ref[0, :, 1:2] = accy
    px = (accx + 1.0) * 0.5 * 64.0 - 0.5
    py = (accy + 1.0) * 0.5 * 64.0 - 0.5
    x0 = jnp.floor(px)
    y0 = jnp.floor(py)
    wx1 = px - x0
    wx0 = 1.0 - wx1
    wy1 = py - y0
    wy0 = 1.0 - wy1
    corners = ((x0, y0, wx0 * wy0), (x0 + 1.0, y0, wx1 * wy0),
               (x0, y0 + 1.0, wx0 * wy1), (x0 + 1.0, y0 + 1.0, wx1 * wy1))
    for ci, (xf, yf, w) in enumerate(corners):
        valid = ((xf >= 0) & (xf < _WS) & (yf >= 0) & (yf < _HS))
        w4_ref[0, :, ci:ci + 1] = w * valid.astype(jnp.float32)
        xc = jnp.clip(xf, 0, _WS - 1).astype(jnp.int32)
        yc = jnp.clip(yf, 0, _HS - 1).astype(jnp.int32)
        cidx_ref[0, ci, :] = jnp.reshape(yc * _WS + xc, (_K,))


_TPB = _N // 16
_CPB = _HW // 16


def _cells_body(loct_ref, cells_ref):
    lx = jnp.clip(loct_ref[0, 0:1, :], -1.0, 1.0)
    ly = jnp.clip(loct_ref[0, 1:2, :], -1.0, 1.0)
    pxf = 0.5 * (lx + 1.0) * 64.0 - 0.5
    pyf = 0.5 * (ly + 1.0) * 64.0 - 0.5
    xi = jnp.clip(jnp.round(pxf).astype(jnp.int32), 0, _WS - 1)
    yi = jnp.clip(jnp.round(pyf).astype(jnp.int32), 0, _HS - 1)
    cells_ref[0] = xi + yi * _WS


def _hist_body(cells_ref, cellst_ref, cnt_ref):
    cell_row = cells_ref[0]
    cell_col = cellst_ref[0]
    y_col = jax.lax.broadcasted_iota(jnp.int32, (_HS, 1), 0)
    x_row = jax.lax.broadcasted_iota(jnp.int32, (1, _WS), 1)
    a = ((cell_row >> 6) == y_col).astype(jnp.bfloat16)
    bm = ((cell_col & 63) == x_row).astype(jnp.bfloat16)
    cnt_ref[0] = jnp.dot(a, bm, preferred_element_type=jnp.float32)


def _sc_scatter_body(src_hbm, cells_hbm, zeros_hbm, out_hbm,
                     idx_v, rows_v, acc_sh):
    c = lax.axis_index("c")
    s = lax.axis_index("s")
    base = s * _TPB
    for bb in range(_B // 2):
        b = c * (_B // 2) + bb
        pltpu.sync_copy(zeros_hbm.at[pl.ds(base, _CPB)],
                        acc_sh.at[pl.ds(base, _CPB)])
        pltpu.sync_copy(src_hbm.at[b, pl.ds(base, _TPB)], rows_v)
        pltpu.sync_copy(cells_hbm.at[b, s], idx_v)
        plsc.subcore_barrier()
        for j in range(_TPB // 128):
            pltpu.sync_copy(rows_v.at[pl.ds(j * 128, 128)],
                            acc_sh.at[idx_v.at[j]], add=True)
        plsc.subcore_barrier()
        pltpu.sync_copy(acc_sh.at[pl.ds(base, _CPB)], rows_v)
        pltpu.sync_copy(rows_v, out_hbm.at[b, pl.ds(base, _CPB)])
        plsc.subcore_barrier()


_GK = None


def _gauss_weights():
    import math as _math
    import numpy as _np
    coords = _np.arange(3, dtype=_np.float32)
    x_grid = _np.tile(coords, 3).reshape(3, 3)
    y_grid = x_grid.T
    mean, variance = 1.0, 4.0
    gk = (1.0 / (2.0 * _math.pi * variance)
          * _np.exp(-((x_grid - mean) ** 2 + (y_grid - mean) ** 2)
                    / (2.0 * variance)))
    gk = gk / gk.sum()
    return gk.astype(_np.float32)


def _blur_body(feat_ref, cnt_ref, out_ref):
    gk = _gauss_weights()
    cnt = cnt_ref[0]
    mask = (cnt > 0).astype(jnp.float32)
    feature = feat_ref[0] / (cnt + 1e-6) * mask
    zf = jnp.zeros((65, _C), jnp.float32)
    zm = jnp.zeros((65, 1), jnp.float32)
    fp = jnp.concatenate([zf, feature, zf], axis=0)
    mp = jnp.concatenate([zm, mask, zm], axis=0)
    xpos = jax.lax.broadcasted_iota(jnp.int32, (_HW, 1), 0) & (_WS - 1)
    accf = jnp.zeros((_HW, _C), jnp.float32)
    accm = jnp.zeros((_HW, 1), jnp.float32)
    for dy in (-1, 0, 1):
        for dx in (-1, 0, 1):
            w = float(gk[dy + 1, dx + 1])
            o = 65 + dy * _WS + dx
            if dx == -1:
                xm = (xpos >= 1).astype(jnp.float32)
            elif dx == 1:
                xm = (xpos <= _WS - 2).astype(jnp.float32)
            else:
                xm = None
            fs = fp[o:o + _HW, :]
            ms = mp[o:o + _HW, :]
            if xm is not None:
                fs = fs * xm
                ms = ms * xm
            accf = accf + w * fs
            accm = accm + w * ms
    fi = accf / (accm + 1e-6)
    mi = (accm > 0).astype(jnp.float32)
    fi = fi * mi
    out_ref[0] = feature + (1.0 - mask) * fi


def _sc_gather_body(xmap_hbm, cidx_hbm, out_hbm, idx_v, rows_v, sem):
    c = lax.axis_index("c")
    s = lax.axis_index("s")
    base = s * _TPB
    for bb in range(_B // 2):
        b = c * (_B // 2) + bb
        pltpu.sync_copy(cidx_hbm.at[b, s], idx_v)
        for j in range(_TPB // 128):
            pltpu.async_copy(xmap_hbm.at[b].at[idx_v.at[j]],
                             rows_v.at[pl.ds(j * 128, 128)], sem).wait()
        pltpu.sync_copy(rows_v, out_hbm.at[b, pl.ds(base, _TPB)])


def _combine_body(rows_ref, w4_ref, ld_ref, xg_ref, lg_ref, pw_ref, pb_ref,
                  out_ref):
    pw0 = pw_ref[0:1, :]
    pw1 = pw_ref[1:2, :]
    pb = pb_ref[...]
    lgx = lg_ref[0, :, 0:1]
    lgy = lg_ref[0, :, 1:2]
    out_ref[0, 0:_NG, :] = xg_ref[0] + (lgx * pw0 + lgy * pw1 + pb)
    acc = jnp.zeros((_K, _C), jnp.float32)
    for ci in range(4):
        acc = acc + w4_ref[0, :, ci:ci + 1] * rows_ref[0, ci]
    lx = ld_ref[0, :, 0:1]
    ly = ld_ref[0, :, 1:2]
    out_ref[0, _NG:, :] = acc + (lx * pw0 + ly * pw1 + pb)


def kernel(x, loc, ln_w, ln_b, W_conf, b_conf, W_pos, b_pos, H, W, N_grid):
    del H, W, N_grid
    f32 = jnp.float32
    x = x.astype(f32)
    loc = loc.astype(f32)

    u = jax.random.uniform(jax.random.key(42), (_B, _NA), dtype=f32)
    nz = -1.0 * jnp.log(u + 1e-6)
    nz = -1.0 * jnp.log(nz + 1e-6)

    loct = jnp.transpose(loc, (0, 2, 1))
    loct_ada = loct[:, :, _NG:]

    mu = jnp.mean(x, axis=-1, keepdims=True)
    var = jnp.mean((x - mu) ** 2, axis=-1, keepdims=True)
    y = (x - mu) / jnp.sqrt(var + 1e-5) * ln_w + ln_b
    conf = y @ W_conf.T + b_conf
    scores = conf[:, _NG:, 0] + nz

    loc_down, cidx, w4 = pl.pallas_call(
        _select_body,
        grid=(_B,),
        in_specs=[
            pl.BlockSpec((1, 1, _NA), lambda b: (b, 0, 0)),
            pl.BlockSpec((1, _NA, 1), lambda b: (b, 0, 0)),
            pl.BlockSpec((1, 2, _NA), lambda b: (b, 0, 0)),
        ],
        out_specs=[
            pl.BlockSpec((1, _K, 2), lambda b: (b, 0, 0)),
            pl.BlockSpec((1, 4, _K), lambda b: (b, 0, 0)),
            pl.BlockSpec((1, _K, 4), lambda b: (b, 0, 0)),
        ],
        out_shape=[
            jax.ShapeDtypeStruct((_B, _K, 2), f32),
            jax.ShapeDtypeStruct((_B, 4, _K), jnp.int32),
            jax.ShapeDtypeStruct((_B, _K, 4), f32),
        ],
    )(scores.reshape(_B, 1, _NA), scores.reshape(_B, _NA, 1), loct_ada)

    cells = pl.pallas_call(
        _cells_body,
        grid=(_B,),
        in_specs=[pl.BlockSpec((1, 2, _N), lambda b: (b, 0, 0))],
        out_specs=pl.BlockSpec((1, 1, _N), lambda b: (b, 0, 0)),
        out_shape=jax.ShapeDtypeStruct((_B, 1, _N), jnp.int32),
    )(loct)

    cnt = pl.pallas_call(
        _hist_body,
        grid=(_B,),
        in_specs=[
            pl.BlockSpec((1, 1, _N), lambda b: (b, 0, 0)),
            pl.BlockSpec((1, _N, 1), lambda b: (b, 0, 0)),
        ],
        out_specs=pl.BlockSpec((1, _HS, _WS), lambda b: (b, 0, 0)),
        out_shape=jax.ShapeDtypeStruct((_B, _HS, _WS), f32),
    )(cells, cells.reshape(_B, _N, 1)).reshape(_B, _HW, 1)

    sc_scatter = functools.partial(
        pl.kernel,
        out_type=jax.ShapeDtypeStruct((_B, _HW, _C), f32),
        mesh=plsc.VectorSubcoreMesh(core_axis_name="c", subcore_axis_name="s"),
        scratch_types=[
            pltpu.VMEM((_TPB // 128, 128), jnp.int32),
            pltpu.VMEM((_TPB, _C), f32),
            pltpu.VMEM_SHARED((_HW, _C), f32),
        ],
    )(_sc_scatter_body)
    feat = sc_scatter(x, cells.reshape(_B, 16, _TPB // 128, 128),
                      jnp.zeros((_HW, _C), f32))

    xmap = pl.pallas_call(
        _blur_body,
        grid=(_B,),
        in_specs=[
            pl.BlockSpec((1, _HW, _C), lambda b: (b, 0, 0)),
            pl.BlockSpec((1, _HW, 1), lambda b: (b, 0, 0)),
        ],
        out_specs=pl.BlockSpec((1, _HW, _C), lambda b: (b, 0, 0)),
        out_shape=jax.ShapeDtypeStruct((_B, _HW, _C), f32),
    )(feat, cnt)

    sc_gather = functools.partial(
        pl.kernel,
        out_type=jax.ShapeDtypeStruct((_B, 4 * _K, _C), f32),
        mesh=plsc.VectorSubcoreMesh(core_axis_name="c", subcore_axis_name="s"),
        scratch_types=[
            pltpu.VMEM((_TPB // 128, 128), jnp.int32),
            pltpu.VMEM((_TPB, _C), f32),
            pltpu.SemaphoreType.DMA,
        ],
    )(_sc_gather_body)
    rows4 = sc_gather(xmap, cidx.reshape(_B, 16, _TPB // 128, 128))

    out = pl.pallas_call(
        _combine_body,
        grid=(_B,),
        in_specs=[
            pl.BlockSpec((1, 4, _K, _C), lambda b: (b, 0, 0, 0)),
            pl.BlockSpec((1, _K, 4), lambda b: (b, 0, 0)),
            pl.BlockSpec((1, _K, 2), lambda b: (b, 0, 0)),
            pl.BlockSpec((1, _NG, _C), lambda b: (b, 0, 0)),
            pl.BlockSpec((1, _NG, 2), lambda b: (b, 0, 0)),
            pl.BlockSpec((2, _C), lambda b: (0, 0)),
            pl.BlockSpec((1, _C), lambda b: (0, 0)),
        ],
        out_specs=pl.BlockSpec((1, 2 * _K, _C), lambda b: (b, 0, 0)),
        out_shape=jax.ShapeDtypeStruct((_B, 2 * _K, _C), f32),
    )(rows4.reshape(_B, 4, _K, _C), w4, loc_down, x[:, :_NG], loc[:, :_NG],
      jnp.transpose(W_pos.astype(f32)), b_pos.reshape(1, _C).astype(f32))

    return out


# speedup vs baseline: 1.2218x; 1.0019x over previous
import functools

import jax
import jax.numpy as jnp
from jax import lax
from jax.experimental import pallas as pl
from jax.experimental.pallas import tpu as pltpu
from jax.experimental.pallas import tpu_sc as plsc

_B, _N, _C = 8, 4096, 128
_NG = 1024
_NA = _N - _NG
_K = 1024
_HS = 64
_WS = 64
_HW = _HS * _WS
_CH = 512


def _select_body(srow_ref, scol_ref, loct_ref, out_ref, cidx_ref, w4_ref):
    rank_rows = []
    for it in range(0, _NA, _CH):
        acc = jnp.zeros((1, _CH), jnp.float32)
        srow = srow_ref[0, 0:1, it:it + _CH]
        iio = jax.lax.broadcasted_iota(jnp.int32, (1, _CH), 1) + it
        for jt in range(0, _NA, _CH):
            scol = scol_ref[0, jt:jt + _CH, :]
            jio = jax.lax.broadcasted_iota(jnp.int32, (_CH, 1), 0) + jt
            gt = (scol > srow) | ((scol == srow) & (jio < iio))
            acc = acc + jnp.sum(gt.astype(jnp.float32), axis=0, keepdims=True)
        rank_rows.append(acc)
    rcol = jax.lax.broadcasted_iota(jnp.int32, (_K, 1), 0).astype(jnp.float32)
    accx = jnp.zeros((_K, 1), jnp.float32)
    accy = jnp.zeros((_K, 1), jnp.float32)
    for t, it in enumerate(range(0, _NA, _CH)):
        oh = (rank_rows[t] == rcol).astype(jnp.float32)
        lx = loct_ref[0, 0:1, it:it + _CH]
        ly = loct_ref[0, 1:2, it:it + _CH]
        accx = accx + jnp.sum(oh * lx, axis=1, keepdims=True)
        accy = accy + jnp.sum(oh * ly, axis=1, keepdims=True)
    out_ref[0, :, 0:1] = accx
    out_ref[0, :, 1:2] = accy
    px = (accx + 1.0) * 0.5 * 64.0 - 0.5
    py = (accy + 1.0) * 0.5 * 64.0 - 0.5
    x0 = jnp.floor(px)
    y0 = jnp.floor(py)
    wx1 = px - x0
    wx0 = 1.0 - wx1
    wy1 = py - y0
    wy0 = 1.0 - wy1
    corners = ((x0, y0, wx0 * wy0), (x0 + 1.0, y0, wx1 * wy0),
               (x0, y0 + 1.0, wx0 * wy1), (x0 + 1.0, y0 + 1.0, wx1 * wy1))
    for ci, (xf, yf, w) in enumerate(corners):
        valid = ((xf >= 0) & (xf < _WS) & (yf >= 0) & (yf < _HS))
        w4_ref[0, :, ci:ci + 1] = w * valid.astype(jnp.float32)
        xc = jnp.clip(xf, 0, _WS - 1).astype(jnp.int32)
        yc = jnp.clip(yf, 0, _HS - 1).astype(jnp.int32)
        cidx_ref[0, ci, :] = jnp.reshape(yc * _WS + xc, (_K,))


_TPB = _N // 16
_CPB = _HW // 16


def _cells_body(loct_ref, cells_ref):
    lx = jnp.clip(loct_ref[0, 0:1, :], -1.0, 1.0)
    ly = jnp.clip(loct_ref[0, 1:2, :], -1.0, 1.0)
    pxf = 0.5 * (lx + 1.0) * 64.0 - 0.5
    pyf = 0.5 * (ly + 1.0) * 64.0 - 0.5
    xi = jnp.clip(jnp.round(pxf).astype(jnp.int32), 0, _WS - 1)
    yi = jnp.clip(jnp.round(pyf).astype(jnp.int32), 0, _HS - 1)
    cells_ref[0] = xi + yi * _WS


def _hist_body(cells_ref, cellst_ref, cnt_ref):
    cell_row = cells_ref[0]
    cell_col = cellst_ref[0]
    y_col = jax.lax.broadcasted_iota(jnp.int32, (_HS, 1), 0)
    x_row = jax.lax.broadcasted_iota(jnp.int32, (1, _WS), 1)
    a = ((cell_row >> 6) == y_col).astype(jnp.bfloat16)
    bm = ((cell_col & 63) == x_row).astype(jnp.bfloat16)
    cnt_ref[0] = jnp.dot(a, bm, preferred_element_type=jnp.float32)


def _sc_scatter_body(src_hbm, cells_hbm, zeros_hbm, out_hbm,
                     idx_v, rows_v, acc_sh):
    c = lax.axis_index("c")
    s = lax.axis_index("s")
    base = s * _TPB
    for bb in range(_B // 2):
        b = c * (_B // 2) + bb
        pltpu.sync_copy(zeros_hbm.at[pl.ds(base, _CPB)],
                        acc_sh.at[pl.ds(base, _CPB)])
        pltpu.sync_copy(src_hbm.at[b, pl.ds(base, _TPB)], rows_v)
        pltpu.sync_copy(cells_hbm.at[b, s], idx_v)
        plsc.subcore_barrier()
        for j in range(_TPB // 128):
            pltpu.sync_copy(rows_v.at[pl.ds(j * 128, 128)],
                            acc_sh.at[idx_v.at[j]], add=True)
        plsc.subcore_barrier()
        pltpu.sync_copy(acc_sh.at[pl.ds(base, _CPB)],
                        out_hbm.at[b, pl.ds(base, _CPB)])
        plsc.subcore_barrier()


_GK = None


def _gauss_weights():
    import math as _math
    import numpy as _np
    coords = _np.arange(3, dtype=_np.float32)
    x_grid = _np.tile(coords, 3).reshape(3, 3)
    y_grid = x_grid.T
    mean, variance = 1.0, 4.0
    gk = (1.0 / (2.0 * _math.pi * variance)
          * _np.exp(-((x_grid - mean) ** 2 + (y_grid - mean) ** 2)
                    / (2.0 * variance)))
    gk = gk / gk.sum()
    return gk.astype(_np.float32)


def _blur_body(feat_ref, cnt_ref, out_ref):
    gk = _gauss_weights()
    cnt = cnt_ref[0]
    mask = (cnt > 0).astype(jnp.float32)
    feature = feat_ref[0] / (cnt + 1e-6) * mask
    zf = jnp.zeros((65, _C), jnp.float32)
    zm = jnp.zeros((65, 1), jnp.float32)
    fp = jnp.concatenate([zf, feature, zf], axis=0)
    mp = jnp.concatenate([zm, mask, zm], axis=0)
    xpos = jax.lax.broadcasted_iota(jnp.int32, (_HW, 1), 0) & (_WS - 1)
    accf = jnp.zeros((_HW, _C), jnp.float32)
    accm = jnp.zeros((_HW, 1), jnp.float32)
    for dy in (-1, 0, 1):
        for dx in (-1, 0, 1):
            w = float(gk[dy + 1, dx + 1])
            o = 65 + dy * _WS + dx
            if dx == -1:
                xm = (xpos >= 1).astype(jnp.float32)
            elif dx == 1:
                xm = (xpos <= _WS - 2).astype(jnp.float32)
            else:
                xm = None
            fs = fp[o:o + _HW, :]
            ms = mp[o:o + _HW, :]
            if xm is not None:
                fs = fs * xm
                ms = ms * xm
            accf = accf + w * fs
            accm = accm + w * ms
    fi = accf / (accm + 1e-6)
    mi = (accm > 0).astype(jnp.float32)
    fi = fi * mi
    out_ref[0] = feature + (1.0 - mask) * fi


def _sc_gather_body(xmap_hbm, cidx_hbm, out_hbm, idx_v, rows_v, sem):
    c = lax.axis_index("c")
    s = lax.axis_index("s")
    base = s * _TPB
    for bb in range(_B // 2):
        b = c * (_B // 2) + bb
        pltpu.sync_copy(cidx_hbm.at[b, s], idx_v)
        for j in range(_TPB // 128):
            pltpu.async_copy(xmap_hbm.at[b].at[idx_v.at[j]],
                             rows_v.at[pl.ds(j * 128, 128)], sem).wait()
        pltpu.sync_copy(rows_v, out_hbm.at[b, pl.ds(base, _TPB)])


def _combine_body(rows_ref, w4_ref, ld_ref, xg_ref, lg_ref, pw_ref, pb_ref,
                  out_ref):
    pw0 = pw_ref[0:1, :]
    pw1 = pw_ref[1:2, :]
    pb = pb_ref[...]
    lgx = lg_ref[0, :, 0:1]
    lgy = lg_ref[0, :, 1:2]
    out_ref[0, 0:_NG, :] = xg_ref[0] + (lgx * pw0 + lgy * pw1 + pb)
    acc = jnp.zeros((_K, _C), jnp.float32)
    for ci in range(4):
        acc = acc + w4_ref[0, :, ci:ci + 1] * rows_ref[0, ci]
    lx = ld_ref[0, :, 0:1]
    ly = ld_ref[0, :, 1:2]
    out_ref[0, _NG:, :] = acc + (lx * pw0 + ly * pw1 + pb)


def kernel(x, loc, ln_w, ln_b, W_conf, b_conf, W_pos, b_pos, H, W, N_grid):
    del H, W, N_grid
    f32 = jnp.float32
    x = x.astype(f32)
    loc = loc.astype(f32)

    u = jax.random.uniform(jax.random.key(42), (_B, _NA), dtype=f32)
    nz = -1.0 * jnp.log(u + 1e-6)
    nz = -1.0 * jnp.log(nz + 1e-6)

    loct = jnp.transpose(loc, (0, 2, 1))
    loct_ada = loct[:, :, _NG:]

    mu = jnp.mean(x, axis=-1, keepdims=True)
    var = jnp.mean((x - mu) ** 2, axis=-1, keepdims=True)
    y = (x - mu) / jnp.sqrt(var + 1e-5) * ln_w + ln_b
    conf = y @ W_conf.T + b_conf
    scores = conf[:, _NG:, 0] + nz

    loc_down, cidx, w4 = pl.pallas_call(
        _select_body,
        grid=(_B,),
        in_specs=[
            pl.BlockSpec((1, 1, _NA), lambda b: (b, 0, 0)),
            pl.BlockSpec((1, _NA, 1), lambda b: (b, 0, 0)),
            pl.BlockSpec((1, 2, _NA), lambda b: (b, 0, 0)),
        ],
        out_specs=[
            pl.BlockSpec((1, _K, 2), lambda b: (b, 0, 0)),
            pl.BlockSpec((1, 4, _K), lambda b: (b, 0, 0)),
            pl.BlockSpec((1, _K, 4), lambda b: (b, 0, 0)),
        ],
        out_shape=[
            jax.ShapeDtypeStruct((_B, _K, 2), f32),
            jax.ShapeDtypeStruct((_B, 4, _K), jnp.int32),
            jax.ShapeDtypeStruct((_B, _K, 4), f32),
        ],
    )(scores.reshape(_B, 1, _NA), scores.reshape(_B, _NA, 1), loct_ada)

    cells = pl.pallas_call(
        _cells_body,
        grid=(_B,),
        in_specs=[pl.BlockSpec((1, 2, _N), lambda b: (b, 0, 0))],
        out_specs=pl.BlockSpec((1, 1, _N), lambda b: (b, 0, 0)),
        out_shape=jax.ShapeDtypeStruct((_B, 1, _N), jnp.int32),
    )(loct)

    cnt = pl.pallas_call(
        _hist_body,
        grid=(_B,),
        in_specs=[
            pl.BlockSpec((1, 1, _N), lambda b: (b, 0, 0)),
            pl.BlockSpec((1, _N, 1), lambda b: (b, 0, 0)),
        ],
        out_specs=pl.BlockSpec((1, _HS, _WS), lambda b: (b, 0, 0)),
        out_shape=jax.ShapeDtypeStruct((_B, _HS, _WS), f32),
    )(cells, cells.reshape(_B, _N, 1)).reshape(_B, _HW, 1)

    sc_scatter = functools.partial(
        pl.kernel,
        out_type=jax.ShapeDtypeStruct((_B, _HW, _C), f32),
        mesh=plsc.VectorSubcoreMesh(core_axis_name="c", subcore_axis_name="s"),
        scratch_types=[
            pltpu.VMEM((_TPB // 128, 128), jnp.int32),
            pltpu.VMEM((_TPB, _C), f32),
            pltpu.VMEM_SHARED((_HW, _C), f32),
        ],
    )(_sc_scatter_body)
    feat = sc_scatter(x, cells.reshape(_B, 16, _TPB // 128, 128),
                      jnp.zeros((_HW, _C), f32))

    xmap = pl.pallas_call(
        _blur_body,
        grid=(_B,),
        in_specs=[
            pl.BlockSpec((1, _HW, _C), lambda b: (b, 0, 0)),
            pl.BlockSpec((1, _HW, 1), lambda b: (b, 0, 0)),
        ],
        out_specs=pl.BlockSpec((1, _HW, _C), lambda b: (b, 0, 0)),
        out_shape=jax.ShapeDtypeStruct((_B, _HW, _C), f32),
    )(feat, cnt)

    sc_gather = functools.partial(
        pl.kernel,
        out_type=jax.ShapeDtypeStruct((_B, 4 * _K, _C), f32),
        mesh=plsc.VectorSubcoreMesh(core_axis_name="c", subcore_axis_name="s"),
        scratch_types=[
            pltpu.VMEM((_TPB // 128, 128), jnp.int32),
            pltpu.VMEM((_TPB, _C), f32),
            pltpu.SemaphoreType.DMA,
        ],
    )(_sc_gather_body)
    rows4 = sc_gather(xmap, cidx.reshape(_B, 16, _TPB // 128, 128))

    out = pl.pallas_call(
        _combine_body,
        grid=(_B,),
        in_specs=[
            pl.BlockSpec((1, 4, _K, _C), lambda b: (b, 0, 0, 0)),
            pl.BlockSpec((1, _K, 4), lambda b: (b, 0, 0)),
            pl.BlockSpec((1, _K, 2), lambda b: (b, 0, 0)),
            pl.BlockSpec((1, _NG, _C), lambda b: (b, 0, 0)),
            pl.BlockSpec((1, _NG, 2), lambda b: (b, 0, 0)),
            pl.BlockSpec((2, _C), lambda b: (0, 0)),
            pl.BlockSpec((1, _C), lambda b: (0, 0)),
        ],
        out_specs=pl.BlockSpec((1, 2 * _K, _C), lambda b: (b, 0, 0)),
        out_shape=jax.ShapeDtypeStruct((_B, 2 * _K, _C), f32),
    )(rows4.reshape(_B, 4, _K, _C), w4, loc_down, x[:, :_NG], loc[:, :_NG],
      jnp.transpose(W_pos.astype(f32)), b_pos.reshape(1, _C).astype(f32))

    return out


# speedup vs baseline: 1.2317x; 1.0081x over previous
import functools

import jax
import jax.numpy as jnp
from jax import lax
from jax.experimental import pallas as pl
from jax.experimental.pallas import tpu as pltpu
from jax.experimental.pallas import tpu_sc as plsc

_B, _N, _C = 8, 4096, 128
_NG = 1024
_NA = _N - _NG
_K = 1024
_HS = 64
_WS = 64
_HW = _HS * _WS
_CH = 512


def _select_body(srow_ref, scol_ref, loct_ref, out_ref, cidx_ref, w4_ref):
    rank_rows = []
    for it in range(0, _NA, _CH):
        acc = jnp.zeros((1, _CH), jnp.float32)
        srow = srow_ref[0, 0:1, it:it + _CH]
        iio = jax.lax.broadcasted_iota(jnp.int32, (1, _CH), 1) + it
        for jt in range(0, _NA, _CH):
            scol = scol_ref[0, jt:jt + _CH, :]
            jio = jax.lax.broadcasted_iota(jnp.int32, (_CH, 1), 0) + jt
            gt = (scol > srow) | ((scol == srow) & (jio < iio))
            acc = acc + jnp.sum(gt.astype(jnp.float32), axis=0, keepdims=True)
        rank_rows.append(acc)
    rcol = jax.lax.broadcasted_iota(jnp.int32, (_K, 1), 0).astype(jnp.float32)
    accx = jnp.zeros((_K, 1), jnp.float32)
    accy = jnp.zeros((_K, 1), jnp.float32)
    for t, it in enumerate(range(0, _NA, _CH)):
        oh = (rank_rows[t] == rcol).astype(jnp.float32)
        lx = loct_ref[0, 0:1, it:it + _CH]
        ly = loct_ref[0, 1:2, it:it + _CH]
        accx = accx + jnp.sum(oh * lx, axis=1, keepdims=True)
        accy = accy + jnp.sum(oh * ly, axis=1, keepdims=True)
    out_ref[0, :, 0:1] = accx
    out_ref[0, :, 1:2] = accy
    px = (accx + 1.0) * 0.5 * 64.0 - 0.5
    py = (accy + 1.0) * 0.5 * 64.0 - 0.5
    x0 = jnp.floor(px)
    y0 = jnp.floor(py)
    wx1 = px - x0
    wx0 = 1.0 - wx1
    wy1 = py - y0
    wy0 = 1.0 - wy1
    corners = ((x0, y0, wx0 * wy0), (x0 + 1.0, y0, wx1 * wy0),
               (x0, y0 + 1.0, wx0 * wy1), (x0 + 1.0, y0 + 1.0, wx1 * wy1))
    for ci, (xf, yf, w) in enumerate(corners):
        valid = ((xf >= 0) & (xf < _WS) & (yf >= 0) & (yf < _HS))
        w4_ref[0, :, ci:ci + 1] = w * valid.astype(jnp.float32)
        xc = jnp.clip(xf, 0, _WS - 1).astype(jnp.int32)
        yc = jnp.clip(yf, 0, _HS - 1).astype(jnp.int32)
        cidx_ref[0, ci, :] = jnp.reshape(yc * _WS + xc, (_K,))


_TPB = _N // 16
_CPB = _HW // 16


def _cells_body(loct_ref, cells_ref):
    lx = jnp.clip(loct_ref[0, 0:1, :], -1.0, 1.0)
    ly = jnp.clip(loct_ref[0, 1:2, :], -1.0, 1.0)
    pxf = 0.5 * (lx + 1.0) * 64.0 - 0.5
    pyf = 0.5 * (ly + 1.0) * 64.0 - 0.5
    xi = jnp.clip(jnp.round(pxf).astype(jnp.int32), 0, _WS - 1)
    yi = jnp.clip(jnp.round(pyf).astype(jnp.int32), 0, _HS - 1)
    cells_ref[0] = xi + yi * _WS


def _hist_body(cells_ref, cellst_ref, cnt_ref):
    cell_row = cells_ref[0]
    cell_col = cellst_ref[0]
    y_col = jax.lax.broadcasted_iota(jnp.int32, (_HS, 1), 0)
    x_row = jax.lax.broadcasted_iota(jnp.int32, (1, _WS), 1)
    a = ((cell_row >> 6) == y_col).astype(jnp.bfloat16)
    bm = ((cell_col & 63) == x_row).astype(jnp.bfloat16)
    cnt_ref[0] = jnp.dot(a, bm, preferred_element_type=jnp.float32)


def _sc_scatter_body(src_hbm, cells_hbm, zeros_hbm, out_hbm,
                     idx_v, rows_v, acc_sh):
    c = lax.axis_index("c")
    s = lax.axis_index("s")
    base = s * _TPB
    for bb in range(_B // 2):
        b = c * (_B // 2) + bb
        pltpu.sync_copy(zeros_hbm.at[pl.ds(base, _CPB)],
                        acc_sh.at[pl.ds(base, _CPB)])
        pltpu.sync_copy(src_hbm.at[b, pl.ds(base, _TPB)], rows_v)
        pltpu.sync_copy(cells_hbm.at[b, s], idx_v)
        plsc.subcore_barrier()
        for j in range(_TPB // 128):
            pltpu.sync_copy(rows_v.at[pl.ds(j * 128, 128)],
                            acc_sh.at[idx_v.at[j]], add=True)
        plsc.subcore_barrier()
        pltpu.sync_copy(acc_sh.at[pl.ds(base, _CPB)],
                        out_hbm.at[b, pl.ds(base, _CPB)])
        plsc.subcore_barrier()


_GK = None


def _gauss_weights():
    import math as _math
    import numpy as _np
    coords = _np.arange(3, dtype=_np.float32)
    x_grid = _np.tile(coords, 3).reshape(3, 3)
    y_grid = x_grid.T
    mean, variance = 1.0, 4.0
    gk = (1.0 / (2.0 * _math.pi * variance)
          * _np.exp(-((x_grid - mean) ** 2 + (y_grid - mean) ** 2)
                    / (2.0 * variance)))
    gk = gk / gk.sum()
    return gk.astype(_np.float32)


def _blur_body(feat_ref, cnt_ref, out_ref):
    gk = _gauss_weights()
    cnt = cnt_ref[0]
    mask = (cnt > 0).astype(jnp.float32)
    feature = feat_ref[0] / (cnt + 1e-6) * mask
    zf = jnp.zeros((65, _C), jnp.bfloat16)
    zm = jnp.zeros((65, 1), jnp.float32)
    fp = jnp.concatenate([zf, feature.astype(jnp.bfloat16), zf], axis=0)
    mp = jnp.concatenate([zm, mask, zm], axis=0)
    xpos = jax.lax.broadcasted_iota(jnp.int32, (_HW, 1), 0) & (_WS - 1)
    accf = jnp.zeros((_HW, _C), jnp.bfloat16)
    accm = jnp.zeros((_HW, 1), jnp.float32)
    for dy in (-1, 0, 1):
        for dx in (-1, 0, 1):
            w = float(gk[dy + 1, dx + 1])
            o = 65 + dy * _WS + dx
            if dx == -1:
                xm = (xpos >= 1)
            elif dx == 1:
                xm = (xpos <= _WS - 2)
            else:
                xm = None
            fs = fp[o:o + _HW, :]
            ms = mp[o:o + _HW, :]
            if xm is not None:
                fs = fs * xm.astype(jnp.bfloat16)
                ms = ms * xm.astype(jnp.float32)
            accf = accf + jnp.bfloat16(w) * fs
            accm = accm + w * ms
    fi = accf.astype(jnp.float32) / (accm + 1e-6)
    mi = (accm > 0).astype(jnp.float32)
    fi = fi * mi
    out_ref[0] = feature + (1.0 - mask) * fi


def _sc_gather_body(xmap_hbm, cidx_hbm, out_hbm, idx_v, rows_v, sem):
    c = lax.axis_index("c")
    s = lax.axis_index("s")
    base = s * _TPB
    for bb in range(_B // 2):
        b = c * (_B // 2) + bb
        pltpu.sync_copy(cidx_hbm.at[b, s], idx_v)
        for j in range(_TPB // 128):
            pltpu.async_copy(xmap_hbm.at[b].at[idx_v.at[j]],
                             rows_v.at[pl.ds(j * 128, 128)], sem).wait()
        pltpu.sync_copy(rows_v, out_hbm.at[b, pl.ds(base, _TPB)])


def _combine_body(rows_ref, w4_ref, ld_ref, xg_ref, lg_ref, pw_ref, pb_ref,
                  out_ref):
    pw0 = pw_ref[0:1, :]
    pw1 = pw_ref[1:2, :]
    pb = pb_ref[...]
    lgx = lg_ref[0, :, 0:1]
    lgy = lg_ref[0, :, 1:2]
    out_ref[0, 0:_NG, :] = xg_ref[0] + (lgx * pw0 + lgy * pw1 + pb)
    acc = jnp.zeros((_K, _C), jnp.float32)
    for ci in range(4):
        acc = acc + w4_ref[0, :, ci:ci + 1] * rows_ref[0, ci]
    lx = ld_ref[0, :, 0:1]
    ly = ld_ref[0, :, 1:2]
    out_ref[0, _NG:, :] = acc + (lx * pw0 + ly * pw1 + pb)


def kernel(x, loc, ln_w, ln_b, W_conf, b_conf, W_pos, b_pos, H, W, N_grid):
    del H, W, N_grid
    f32 = jnp.float32
    x = x.astype(f32)
    loc = loc.astype(f32)

    u = jax.random.uniform(jax.random.key(42), (_B, _NA), dtype=f32)
    nz = -1.0 * jnp.log(u + 1e-6)
    nz = -1.0 * jnp.log(nz + 1e-6)

    loct = jnp.transpose(loc, (0, 2, 1))
    loct_ada = loct[:, :, _NG:]

    mu = jnp.mean(x, axis=-1, keepdims=True)
    var = jnp.mean((x - mu) ** 2, axis=-1, keepdims=True)
    y = (x - mu) / jnp.sqrt(var + 1e-5) * ln_w + ln_b
    conf = y @ W_conf.T + b_conf
    scores = conf[:, _NG:, 0] + nz

    loc_down, cidx, w4 = pl.pallas_call(
        _select_body,
        grid=(_B,),
        in_specs=[
            pl.BlockSpec((1, 1, _NA), lambda b: (b, 0, 0)),
            pl.BlockSpec((1, _NA, 1), lambda b: (b, 0, 0)),
            pl.BlockSpec((1, 2, _NA), lambda b: (b, 0, 0)),
        ],
        out_specs=[
            pl.BlockSpec((1, _K, 2), lambda b: (b, 0, 0)),
            pl.BlockSpec((1, 4, _K), lambda b: (b, 0, 0)),
            pl.BlockSpec((1, _K, 4), lambda b: (b, 0, 0)),
        ],
        out_shape=[
            jax.ShapeDtypeStruct((_B, _K, 2), f32),
            jax.ShapeDtypeStruct((_B, 4, _K), jnp.int32),
            jax.ShapeDtypeStruct((_B, _K, 4), f32),
        ],
    )(scores.reshape(_B, 1, _NA), scores.reshape(_B, _NA, 1), loct_ada)

    cells = pl.pallas_call(
        _cells_body,
        grid=(_B,),
        in_specs=[pl.BlockSpec((1, 2, _N), lambda b: (b, 0, 0))],
        out_specs=pl.BlockSpec((1, 1, _N), lambda b: (b, 0, 0)),
        out_shape=jax.ShapeDtypeStruct((_B, 1, _N), jnp.int32),
    )(loct)

    cnt = pl.pallas_call(
        _hist_body,
        grid=(_B,),
        in_specs=[
            pl.BlockSpec((1, 1, _N), lambda b: (b, 0, 0)),
            pl.BlockSpec((1, _N, 1), lambda b: (b, 0, 0)),
        ],
        out_specs=pl.BlockSpec((1, _HS, _WS), lambda b: (b, 0, 0)),
        out_shape=jax.ShapeDtypeStruct((_B, _HS, _WS), f32),
    )(cells, cells.reshape(_B, _N, 1)).reshape(_B, _HW, 1)

    sc_scatter = functools.partial(
        pl.kernel,
        out_type=jax.ShapeDtypeStruct((_B, _HW, _C), f32),
        mesh=plsc.VectorSubcoreMesh(core_axis_name="c", subcore_axis_name="s"),
        scratch_types=[
            pltpu.VMEM((_TPB // 128, 128), jnp.int32),
            pltpu.VMEM((_TPB, _C), f32),
            pltpu.VMEM_SHARED((_HW, _C), f32),
        ],
    )(_sc_scatter_body)
    feat = sc_scatter(x, cells.reshape(_B, 16, _TPB // 128, 128),
                      jnp.zeros((_HW, _C), f32))

    xmap = pl.pallas_call(
        _blur_body,
        grid=(_B,),
        in_specs=[
            pl.BlockSpec((1, _HW, _C), lambda b: (b, 0, 0)),
            pl.BlockSpec((1, _HW, 1), lambda b: (b, 0, 0)),
        ],
        out_specs=pl.BlockSpec((1, _HW, _C), lambda b: (b, 0, 0)),
        out_shape=jax.ShapeDtypeStruct((_B, _HW, _C), f32),
    )(feat, cnt)

    sc_gather = functools.partial(
        pl.kernel,
        out_type=jax.ShapeDtypeStruct((_B, 4 * _K, _C), f32),
        mesh=plsc.VectorSubcoreMesh(core_axis_name="c", subcore_axis_name="s"),
        scratch_types=[
            pltpu.VMEM((_TPB // 128, 128), jnp.int32),
            pltpu.VMEM((_TPB, _C), f32),
            pltpu.SemaphoreType.DMA,
        ],
    )(_sc_gather_body)
    rows4 = sc_gather(xmap, cidx.reshape(_B, 16, _TPB // 128, 128))

    out = pl.pallas_call(
        _combine_body,
        grid=(_B,),
        in_specs=[
            pl.BlockSpec((1, 4, _K, _C), lambda b: (b, 0, 0, 0)),
            pl.BlockSpec((1, _K, 4), lambda b: (b, 0, 0)),
            pl.BlockSpec((1, _K, 2), lambda b: (b, 0, 0)),
            pl.BlockSpec((1, _NG, _C), lambda b: (b, 0, 0)),
            pl.BlockSpec((1, _NG, 2), lambda b: (b, 0, 0)),
            pl.BlockSpec((2, _C), lambda b: (0, 0)),
            pl.BlockSpec((1, _C), lambda b: (0, 0)),
        ],
        out_specs=pl.BlockSpec((1, 2 * _K, _C), lambda b: (b, 0, 0)),
        out_shape=jax.ShapeDtypeStruct((_B, 2 * _K, _C), f32),
    )(rows4.reshape(_B, 4, _K, _C), w4, loc_down, x[:, :_NG], loc[:, :_NG],
      jnp.transpose(W_pos.astype(f32)), b_pos.reshape(1, _C).astype(f32))

    return out


# speedup vs baseline: 1.2787x; 1.0382x over previous
import functools

import jax
import jax.numpy as jnp
from jax import lax
from jax.experimental import pallas as pl
from jax.experimental.pallas import tpu as pltpu
from jax.experimental.pallas import tpu_sc as plsc

_B, _N, _C = 8, 4096, 128
_NG = 1024
_NA = _N - _NG
_K = 1024
_HS = 64
_WS = 64
_HW = _HS * _WS
_CH = 512


def _select_body(srow_ref, scol_ref, loct_ref, out_ref, cidx_ref, w4_ref):
    nt_ = _NA // _CH
    racc_r = [jnp.zeros((1, _CH), jnp.float32) for _ in range(nt_)]
    racc_c = [jnp.zeros((_CH, 1), jnp.float32) for _ in range(nt_)]
    for it in range(nt_):
        srow = srow_ref[0, 0:1, it * _CH:(it + 1) * _CH]
        iio = jax.lax.broadcasted_iota(jnp.int32, (1, _CH), 1) + it * _CH
        for jt in range(it, nt_):
            scol = scol_ref[0, jt * _CH:(jt + 1) * _CH, :]
            jio = (jax.lax.broadcasted_iota(jnp.int32, (_CH, 1), 0)
                   + jt * _CH)
            gt = ((scol > srow) | ((scol == srow) & (jio < iio))
                  ).astype(jnp.float32)
            racc_r[it] = racc_r[it] + jnp.sum(gt, axis=0, keepdims=True)
            if jt > it:
                racc_c[jt] = racc_c[jt] + (
                    jnp.float32(_CH) - jnp.sum(gt, axis=1, keepdims=True))
    rank_rows = [racc_r[t] + jnp.reshape(racc_c[t], (1, _CH))
                 for t in range(nt_)]
    rcol = jax.lax.broadcasted_iota(jnp.int32, (_K, 1), 0).astype(jnp.float32)
    accx = jnp.zeros((_K, 1), jnp.float32)
    accy = jnp.zeros((_K, 1), jnp.float32)
    for t, it in enumerate(range(0, _NA, _CH)):
        oh = (rank_rows[t] == rcol).astype(jnp.float32)
        lx = loct_ref[0, 0:1, it:it + _CH]
        ly = loct_ref[0, 1:2, it:it + _CH]
        accx = accx + jnp.sum(oh * lx, axis=1, keepdims=True)
        accy = accy + jnp.sum(oh * ly, axis=1, keepdims=True)
    out_ref[0, :, 0:1] = accx
    out_ref[0, :, 1:2] = accy
    px = (accx + 1.0) * 0.5 * 64.0 - 0.5
    py = (accy + 1.0) * 0.5 * 64.0 - 0.5
    x0 = jnp.floor(px)
    y0 = jnp.floor(py)
    wx1 = px - x0
    wx0 = 1.0 - wx1
    wy1 = py - y0
    wy0 = 1.0 - wy1
    corners = ((x0, y0, wx0 * wy0), (x0 + 1.0, y0, wx1 * wy0),
               (x0, y0 + 1.0, wx0 * wy1), (x0 + 1.0, y0 + 1.0, wx1 * wy1))
    for ci, (xf, yf, w) in enumerate(corners):
        valid = ((xf >= 0) & (xf < _WS) & (yf >= 0) & (yf < _HS))
        w4_ref[0, :, ci:ci + 1] = w * valid.astype(jnp.float32)
        xc = jnp.clip(xf, 0, _WS - 1).astype(jnp.int32)
        yc = jnp.clip(yf, 0, _HS - 1).astype(jnp.int32)
        cidx_ref[0, ci, :] = jnp.reshape(yc * _WS + xc, (_K,))


_TPB = _N // 16
_CPB = _HW // 16


def _cells_body(loct_ref, cells_ref):
    lx = jnp.clip(loct_ref[0, 0:1, :], -1.0, 1.0)
    ly = jnp.clip(loct_ref[0, 1:2, :], -1.0, 1.0)
    pxf = 0.5 * (lx + 1.0) * 64.0 - 0.5
    pyf = 0.5 * (ly + 1.0) * 64.0 - 0.5
    xi = jnp.clip(jnp.round(pxf).astype(jnp.int32), 0, _WS - 1)
    yi = jnp.clip(jnp.round(pyf).astype(jnp.int32), 0, _HS - 1)
    cells_ref[0] = xi + yi * _WS


def _hist_body(cells_ref, cellst_ref, cnt_ref):
    cell_row = cells_ref[0]
    cell_col = cellst_ref[0]
    y_col = jax.lax.broadcasted_iota(jnp.int32, (_HS, 1), 0)
    x_row = jax.lax.broadcasted_iota(jnp.int32, (1, _WS), 1)
    a = ((cell_row >> 6) == y_col).astype(jnp.bfloat16)
    bm = ((cell_col & 63) == x_row).astype(jnp.bfloat16)
    cnt_ref[0] = jnp.dot(a, bm, preferred_element_type=jnp.float32)


def _sc_scatter_body(src_hbm, cells_hbm, zeros_hbm, out_hbm,
                     idx_v, rows_v, acc_sh):
    c = lax.axis_index("c")
    s = lax.axis_index("s")
    base = s * _TPB
    for bb in range(_B // 2):
        b = c * (_B // 2) + bb
        pltpu.sync_copy(zeros_hbm.at[pl.ds(base, _CPB)],
                        acc_sh.at[pl.ds(base, _CPB)])
        pltpu.sync_copy(src_hbm.at[b, pl.ds(base, _TPB)], rows_v)
        pltpu.sync_copy(cells_hbm.at[b, s], idx_v)
        plsc.subcore_barrier()
        for j in range(_TPB // 128):
            pltpu.sync_copy(rows_v.at[pl.ds(j * 128, 128)],
                            acc_sh.at[idx_v.at[j]], add=True)
        plsc.subcore_barrier()
        pltpu.sync_copy(acc_sh.at[pl.ds(base, _CPB)],
                        out_hbm.at[b, pl.ds(base, _CPB)])
        plsc.subcore_barrier()


_GK = None


def _gauss_weights():
    import math as _math
    import numpy as _np
    coords = _np.arange(3, dtype=_np.float32)
    x_grid = _np.tile(coords, 3).reshape(3, 3)
    y_grid = x_grid.T
    mean, variance = 1.0, 4.0
    gk = (1.0 / (2.0 * _math.pi * variance)
          * _np.exp(-((x_grid - mean) ** 2 + (y_grid - mean) ** 2)
                    / (2.0 * variance)))
    gk = gk / gk.sum()
    return gk.astype(_np.float32)


def _blur_body(feat_ref, cnt_ref, out_ref):
    gk = _gauss_weights()
    cnt = cnt_ref[0]
    mask = (cnt > 0).astype(jnp.float32)
    feature = feat_ref[0] / (cnt + 1e-6) * mask
    zf = jnp.zeros((65, _C), jnp.bfloat16)
    zm = jnp.zeros((65, 1), jnp.float32)
    fp = jnp.concatenate([zf, feature.astype(jnp.bfloat16), zf], axis=0)
    mp = jnp.concatenate([zm, mask, zm], axis=0)
    xpos = jax.lax.broadcasted_iota(jnp.int32, (_HW, 1), 0) & (_WS - 1)
    accf = jnp.zeros((_HW, _C), jnp.bfloat16)
    accm = jnp.zeros((_HW, 1), jnp.float32)
    for dy in (-1, 0, 1):
        for dx in (-1, 0, 1):
            w = float(gk[dy + 1, dx + 1])
            o = 65 + dy * _WS + dx
            if dx == -1:
                xm = (xpos >= 1)
            elif dx == 1:
                xm = (xpos <= _WS - 2)
            else:
                xm = None
            fs = fp[o:o + _HW, :]
            ms = mp[o:o + _HW, :]
            if xm is not None:
                fs = fs * xm.astype(jnp.bfloat16)
                ms = ms * xm.astype(jnp.float32)
            accf = accf + jnp.bfloat16(w) * fs
            accm = accm + w * ms
    fi = accf.astype(jnp.float32) / (accm + 1e-6)
    mi = (accm > 0).astype(jnp.float32)
    fi = fi * mi
    out_ref[0] = feature + (1.0 - mask) * fi


def _sc_gather_body(xmap_hbm, cidx_hbm, out_hbm, idx_v, rows_v, sem):
    c = lax.axis_index("c")
    s = lax.axis_index("s")
    base = s * _TPB
    for bb in range(_B // 2):
        b = c * (_B // 2) + bb
        pltpu.sync_copy(cidx_hbm.at[b, s], idx_v)
        for j in range(_TPB // 128):
            pltpu.async_copy(xmap_hbm.at[b].at[idx_v.at[j]],
                             rows_v.at[pl.ds(j * 128, 128)], sem).wait()
        pltpu.sync_copy(rows_v, out_hbm.at[b, pl.ds(base, _TPB)])


def _combine_body(rows_ref, w4_ref, ld_ref, xg_ref, lg_ref, pw_ref, pb_ref,
                  out_ref):
    pw0 = pw_ref[0:1, :]
    pw1 = pw_ref[1:2, :]
    pb = pb_ref[...]
    lgx = lg_ref[0, :, 0:1]
    lgy = lg_ref[0, :, 1:2]
    out_ref[0, 0:_NG, :] = xg_ref[0] + (lgx * pw0 + lgy * pw1 + pb)
    acc = jnp.zeros((_K, _C), jnp.float32)
    for ci in range(4):
        acc = acc + w4_ref[0, :, ci:ci + 1] * rows_ref[0, ci]
    lx = ld_ref[0, :, 0:1]
    ly = ld_ref[0, :, 1:2]
    out_ref[0, _NG:, :] = acc + (lx * pw0 + ly * pw1 + pb)


def kernel(x, loc, ln_w, ln_b, W_conf, b_conf, W_pos, b_pos, H, W, N_grid):
    del H, W, N_grid
    f32 = jnp.float32
    x = x.astype(f32)
    loc = loc.astype(f32)

    u = jax.random.uniform(jax.random.key(42), (_B, _NA), dtype=f32)
    nz = -1.0 * jnp.log(u + 1e-6)
    nz = -1.0 * jnp.log(nz + 1e-6)

    loct = jnp.transpose(loc, (0, 2, 1))
    loct_ada = loct[:, :, _NG:]

    mu = jnp.mean(x, axis=-1, keepdims=True)
    var = jnp.mean((x - mu) ** 2, axis=-1, keepdims=True)
    y = (x - mu) / jnp.sqrt(var + 1e-5) * ln_w + ln_b
    conf = y @ W_conf.T + b_conf
    scores = conf[:, _NG:, 0] + nz

    loc_down, cidx, w4 = pl.pallas_call(
        _select_body,
        grid=(_B,),
        in_specs=[
            pl.BlockSpec((1, 1, _NA), lambda b: (b, 0, 0)),
            pl.BlockSpec((1, _NA, 1), lambda b: (b, 0, 0)),
            pl.BlockSpec((1, 2, _NA), lambda b: (b, 0, 0)),
        ],
        out_specs=[
            pl.BlockSpec((1, _K, 2), lambda b: (b, 0, 0)),
            pl.BlockSpec((1, 4, _K), lambda b: (b, 0, 0)),
            pl.BlockSpec((1, _K, 4), lambda b: (b, 0, 0)),
        ],
        out_shape=[
            jax.ShapeDtypeStruct((_B, _K, 2), f32),
            jax.ShapeDtypeStruct((_B, 4, _K), jnp.int32),
            jax.ShapeDtypeStruct((_B, _K, 4), f32),
        ],
    )(scores.reshape(_B, 1, _NA), scores.reshape(_B, _NA, 1), loct_ada)

    cells = pl.pallas_call(
        _cells_body,
        grid=(_B,),
        in_specs=[pl.BlockSpec((1, 2, _N), lambda b: (b, 0, 0))],
        out_specs=pl.BlockSpec((1, 1, _N), lambda b: (b, 0, 0)),
        out_shape=jax.ShapeDtypeStruct((_B, 1, _N), jnp.int32),
    )(loct)

    cnt = pl.pallas_call(
        _hist_body,
        grid=(_B,),
        in_specs=[
            pl.BlockSpec((1, 1, _N), lambda b: (b, 0, 0)),
            pl.BlockSpec((1, _N, 1), lambda b: (b, 0, 0)),
        ],
        out_specs=pl.BlockSpec((1, _HS, _WS), lambda b: (b, 0, 0)),
        out_shape=jax.ShapeDtypeStruct((_B, _HS, _WS), f32),
    )(cells, cells.reshape(_B, _N, 1)).reshape(_B, _HW, 1)

    sc_scatter = functools.partial(
        pl.kernel,
        out_type=jax.ShapeDtypeStruct((_B, _HW, _C), f32),
        mesh=plsc.VectorSubcoreMesh(core_axis_name="c", subcore_axis_name="s"),
        scratch_types=[
            pltpu.VMEM((_TPB // 128, 128), jnp.int32),
            pltpu.VMEM((_TPB, _C), f32),
            pltpu.VMEM_SHARED((_HW, _C), f32),
        ],
    )(_sc_scatter_body)
    feat = sc_scatter(x, cells.reshape(_B, 16, _TPB // 128, 128),
                      jnp.zeros((_HW, _C), f32))

    xmap = pl.pallas_call(
        _blur_body,
        grid=(_B,),
        in_specs=[
            pl.BlockSpec((1, _HW, _C), lambda b: (b, 0, 0)),
            pl.BlockSpec((1, _HW, 1), lambda b: (b, 0, 0)),
        ],
        out_specs=pl.BlockSpec((1, _HW, _C), lambda b: (b, 0, 0)),
        out_shape=jax.ShapeDtypeStruct((_B, _HW, _C), f32),
    )(feat, cnt)

    sc_gather = functools.partial(
        pl.kernel,
        out_type=jax.ShapeDtypeStruct((_B, 4 * _K, _C), f32),
        mesh=plsc.VectorSubcoreMesh(core_axis_name="c", subcore_axis_name="s"),
        scratch_types=[
            pltpu.VMEM((_TPB // 128, 128), jnp.int32),
            pltpu.VMEM((_TPB, _C), f32),
            pltpu.SemaphoreType.DMA,
        ],
    )(_sc_gather_body)
    rows4 = sc_gather(xmap, cidx.reshape(_B, 16, _TPB // 128, 128))

    out = pl.pallas_call(
        _combine_body,
        grid=(_B,),
        in_specs=[
            pl.BlockSpec((1, 4, _K, _C), lambda b: (b, 0, 0, 0)),
            pl.BlockSpec((1, _K, 4), lambda b: (b, 0, 0)),
            pl.BlockSpec((1, _K, 2), lambda b: (b, 0, 0)),
            pl.BlockSpec((1, _NG, _C), lambda b: (b, 0, 0)),
            pl.BlockSpec((1, _NG, 2), lambda b: (b, 0, 0)),
            pl.BlockSpec((2, _C), lambda b: (0, 0)),
            pl.BlockSpec((1, _C), lambda b: (0, 0)),
        ],
        out_specs=pl.BlockSpec((1, 2 * _K, _C), lambda b: (b, 0, 0)),
        out_shape=jax.ShapeDtypeStruct((_B, 2 * _K, _C), f32),
    )(rows4.reshape(_B, 4, _K, _C), w4, loc_down, x[:, :_NG], loc[:, :_NG],
      jnp.transpose(W_pos.astype(f32)), b_pos.reshape(1, _C).astype(f32))

    return out
